# Initial kernel scaffold; baseline (speedup 1.0000x reference)
import jax
import jax.numpy as jnp
from jax.experimental import pallas as pl


def kernel(x, edge_index, batch, W1, b1, W2, b2, Wl1, bl1, Wl2, bl2):
    raise NotImplementedError("write your pallas kernel here")


# speedup vs baseline: 15.7694x; 15.7694x over previous
import functools

import jax
import jax.numpy as jnp
from jax import lax
from jax.experimental import pallas as pl
from jax.experimental.pallas import tpu as pltpu
from jax.experimental.pallas import tpu_sc as plsc

N = 10000
E = 320000
DIN = 128
H = 64
G = 128

NC = 2
NS = 16
NW = NC * NS
CHUNK = 128
CPW = -(-E // (NW * CHUNK))
EPAD = NW * CPW * CHUNK
ROWS = 10240
RPT = ROWS // NS

_MESH = dict(core_axis_name="c", subcore_axis_name="s",
             num_cores=NC, num_subcores=NS)


def _zero_f32_ref(ref, n):
    for k in range(n // 16):
        ref[pl.ds(16 * k, 16)] = jnp.zeros((16,), jnp.float32)


def _deg_body(dst_hbm, out_hbm, dbuf, ones_v, zbuf, acc):
    c = lax.axis_index("c")
    s = lax.axis_index("s")
    w = c * NS + s

    _zero_f32_ref(zbuf, RPT)
    for k in range(CHUNK // 16):
        ones_v[pl.ds(16 * k, 16)] = jnp.ones((16,), jnp.float32)
    pltpu.sync_copy(zbuf, acc.at[pl.ds(s * RPT, RPT)])
    plsc.subcore_barrier()

    ebase = w * CPW * CHUNK

    def body(j, carry):
        off = ebase + j * CHUNK
        pltpu.sync_copy(dst_hbm.at[pl.ds(off, CHUNK)], dbuf.at[0])
        pltpu.sync_copy(ones_v, acc.at[dbuf.at[0]], add=True)
        return carry

    lax.fori_loop(0, CPW, body, 0)
    plsc.subcore_barrier()
    pltpu.sync_copy(acc.at[pl.ds(s * RPT, RPT)],
                    out_hbm.at[c].at[pl.ds(s * RPT, RPT)])


@functools.partial(jax.jit)
def _deg_call(dst_p):
    return pl.kernel(
        _deg_body,
        out_type=jax.ShapeDtypeStruct((NC, ROWS), jnp.float32),
        mesh=plsc.VectorSubcoreMesh(**_MESH),
        scratch_types=[
            pltpu.VMEM((1, CHUNK), jnp.int32),
            pltpu.VMEM((CHUNK,), jnp.float32),
            pltpu.VMEM((RPT,), jnp.float32),
            pltpu.VMEM_SHARED((ROWS,), jnp.float32),
        ],
    )(dst_p)


def _scatter_body(y_hbm, src_hbm, dst_hbm, out_hbm, sbuf, dbuf, rbuf, zbuf,
                  acc):
    c = lax.axis_index("c")
    s = lax.axis_index("s")
    w = c * NS + s

    for r in range(16):
        for k in range(H // 16):
            zbuf[r, pl.ds(16 * k, 16)] = jnp.zeros((16,), jnp.float32)

    def zloop(j, carry):
        pltpu.sync_copy(zbuf, acc.at[pl.ds(s * RPT + 16 * j, 16)])
        return carry

    lax.fori_loop(0, RPT // 16, zloop, 0)
    plsc.subcore_barrier()

    ebase = w * CPW * CHUNK

    def body(j, carry):
        off = ebase + j * CHUNK
        pltpu.sync_copy(src_hbm.at[pl.ds(off, CHUNK)], sbuf.at[0])
        pltpu.sync_copy(dst_hbm.at[pl.ds(off, CHUNK)], dbuf.at[0])
        pltpu.sync_copy(y_hbm.at[sbuf.at[0]], rbuf)
        pltpu.sync_copy(rbuf, acc.at[dbuf.at[0]], add=True)
        return carry

    lax.fori_loop(0, CPW, body, 0)
    plsc.subcore_barrier()
    pltpu.sync_copy(acc.at[pl.ds(s * RPT, RPT)],
                    out_hbm.at[c].at[pl.ds(s * RPT, RPT)])


@functools.partial(jax.jit)
def _scatter_call(y, src_p, dst_p):
    return pl.kernel(
        _scatter_body,
        out_type=jax.ShapeDtypeStruct((NC, ROWS, H), jnp.float32),
        mesh=plsc.VectorSubcoreMesh(**_MESH),
        scratch_types=[
            pltpu.VMEM((1, CHUNK), jnp.int32),
            pltpu.VMEM((1, CHUNK), jnp.int32),
            pltpu.VMEM((CHUNK, H), jnp.float32),
            pltpu.VMEM((16, H), jnp.float32),
            pltpu.VMEM_SHARED((ROWS, H), jnp.float32),
        ],
        compiler_params=pltpu.CompilerParams(use_tc_tiling_on_sc=False),
    )(y, src_p, dst_p)


RB = 2000
NBLK = N // RB


def _t1_body(x_ref, w1_ref, degt_ref, y_ref, z_ref):
    deg = degt_ref[:, 0:1] + degt_ref[:, 1:2] + 1.0
    dis = lax.rsqrt(deg)
    xw = jnp.dot(x_ref[...], w1_ref[...], preferred_element_type=jnp.float32)
    y_ref[...] = xw * dis
    z_ref[...] = xw / deg


def _t1(x, W1, degt):
    return pl.pallas_call(
        _t1_body,
        grid=(NBLK,),
        in_specs=[
            pl.BlockSpec((RB, DIN), lambda i: (i, 0)),
            pl.BlockSpec((DIN, H), lambda i: (0, 0)),
            pl.BlockSpec((RB, 2), lambda i: (i, 0)),
        ],
        out_specs=[
            pl.BlockSpec((RB, H), lambda i: (i, 0)),
            pl.BlockSpec((RB, H), lambda i: (i, 0)),
        ],
        out_shape=[
            jax.ShapeDtypeStruct((N, H), jnp.float32),
            jax.ShapeDtypeStruct((N, H), jnp.float32),
        ],
    )(x, W1, degt)


def _t2_body(acc_ref, z1_ref, degt_ref, w2_ref, b1_ref, y_ref, z_ref):
    deg = degt_ref[:, 0:1] + degt_ref[:, 1:2] + 1.0
    dis = lax.rsqrt(deg)
    a = acc_ref[0] + acc_ref[1]
    h1 = jnp.maximum(a * dis + z1_ref[...] + b1_ref[...], 0.0)
    xw = jnp.dot(h1, w2_ref[...], preferred_element_type=jnp.float32)
    y_ref[...] = xw * dis
    z_ref[...] = xw / deg


def _t2(acc1, z1, degt, W2, b1r):
    return pl.pallas_call(
        _t2_body,
        grid=(NBLK,),
        in_specs=[
            pl.BlockSpec((NC, RB, H), lambda i: (0, i, 0)),
            pl.BlockSpec((RB, H), lambda i: (i, 0)),
            pl.BlockSpec((RB, 2), lambda i: (i, 0)),
            pl.BlockSpec((H, H), lambda i: (0, 0)),
            pl.BlockSpec((1, H), lambda i: (0, 0)),
        ],
        out_specs=[
            pl.BlockSpec((RB, H), lambda i: (i, 0)),
            pl.BlockSpec((RB, H), lambda i: (i, 0)),
        ],
        out_shape=[
            jax.ShapeDtypeStruct((N, H), jnp.float32),
            jax.ShapeDtypeStruct((N, H), jnp.float32),
        ],
    )(acc1, z1, degt, W2, b1r)


def _t3_body(acc_ref, z2_ref, degt_ref, bcol_ref, b2_ref, wl1_ref, bl1_ref,
             wl2_ref, bl2_ref, out_ref, gsum, cnt):
    i = pl.program_id(0)

    @pl.when(i == 0)
    def _():
        gsum[...] = jnp.zeros((G, H), jnp.float32)
        cnt[...] = jnp.zeros((G, 1), jnp.float32)

    deg = degt_ref[:, 0:1] + degt_ref[:, 1:2] + 1.0
    dis = lax.rsqrt(deg)
    h2 = (acc_ref[0] + acc_ref[1]) * dis + z2_ref[...] + b2_ref[...]
    pt = (bcol_ref[...] == lax.broadcasted_iota(jnp.int32, (1, G), 1))
    pt = pt.astype(jnp.float32)
    dn = (((0,), (0,)), ((), ()))
    gsum[...] += lax.dot_general(pt, h2, dn,
                                 preferred_element_type=jnp.float32)
    cnt[...] += lax.dot_general(pt, jnp.ones((RB, 1), jnp.float32), dn,
                                preferred_element_type=jnp.float32)

    @pl.when(i == NBLK - 1)
    def _():
        g = gsum[...] / jnp.maximum(cnt[...], 1.0)
        g = jnp.dot(g, wl1_ref[...],
                    preferred_element_type=jnp.float32) + bl1_ref[...]
        g = jnp.dot(g, wl2_ref[...],
                    preferred_element_type=jnp.float32) + bl2_ref[...]
        m = jnp.max(g, axis=0, keepdims=True)
        e = jnp.exp(g - m)
        out_ref[...] = e / jnp.sum(e, axis=0, keepdims=True)


def _t3(acc2, z2, degt, bcol, b2r, Wl1, bl1r, Wl2, bl2r):
    return pl.pallas_call(
        _t3_body,
        grid=(NBLK,),
        in_specs=[
            pl.BlockSpec((NC, RB, H), lambda i: (0, i, 0)),
            pl.BlockSpec((RB, H), lambda i: (i, 0)),
            pl.BlockSpec((RB, 2), lambda i: (i, 0)),
            pl.BlockSpec((RB, 1), lambda i: (i, 0)),
            pl.BlockSpec((1, H), lambda i: (0, 0)),
            pl.BlockSpec((H, 32), lambda i: (0, 0)),
            pl.BlockSpec((1, 32), lambda i: (0, 0)),
            pl.BlockSpec((32, 2), lambda i: (0, 0)),
            pl.BlockSpec((1, 2), lambda i: (0, 0)),
        ],
        out_specs=pl.BlockSpec((G, 2), lambda i: (0, 0)),
        out_shape=jax.ShapeDtypeStruct((G, 2), jnp.float32),
        scratch_shapes=[
            pltpu.VMEM((G, H), jnp.float32),
            pltpu.VMEM((G, 1), jnp.float32),
        ],
    )(acc2, z2, degt, bcol, b2r, Wl1, bl1r, Wl2, bl2r)


def kernel(x, edge_index, batch, W1, b1, W2, b2, Wl1, bl1, Wl2, bl2):
    src = edge_index[0].astype(jnp.int32)
    dst = edge_index[1].astype(jnp.int32)
    pad = EPAD - E
    src_p = jnp.concatenate([src, jnp.zeros((pad,), jnp.int32)])
    dst_p = jnp.concatenate([dst, jnp.full((pad,), N, jnp.int32)])

    degp = _deg_call(dst_p)
    degt = degp.T[:N]

    y1, z1 = _t1(x, W1, degt)
    acc1 = _scatter_call(y1, src_p, dst_p)
    y2, z2 = _t2(acc1, z1, degt, W2, b1.reshape(1, H))
    acc2 = _scatter_call(y2, src_p, dst_p)
    return _t3(acc2, z2, degt, batch.astype(jnp.int32).reshape(N, 1),
               b2.reshape(1, H), Wl1, bl1.reshape(1, 32),
               Wl2, bl2.reshape(1, 2))


# speedup vs baseline: 15.7732x; 1.0002x over previous
import functools

import jax
import jax.numpy as jnp
from jax import lax
from jax.experimental import pallas as pl
from jax.experimental.pallas import tpu as pltpu
from jax.experimental.pallas import tpu_sc as plsc

N = 10000
E = 320000
DIN = 128
H = 64
G = 128

NC = 2
NS = 16
NW = NC * NS
CHUNK = 128
CPW = 80
NCH = NW * CPW
EPAD = NCH * CHUNK
ROWS = 10240
RPT = ROWS // NS

_MESH = dict(core_axis_name="c", subcore_axis_name="s",
             num_cores=NC, num_subcores=NS)


def _zero_f32_ref(ref, n):
    for k in range(n // 16):
        ref[pl.ds(16 * k, 16)] = jnp.zeros((16,), jnp.float32)


def _deg_body(ep_hbm, out_hbm, ib0, ib1, ones_v, zbuf, acc, sem0, sem1):
    c = lax.axis_index("c")
    s = lax.axis_index("s")
    w = c * NS + s

    _zero_f32_ref(zbuf, RPT)
    for k in range(CHUNK // 16):
        ones_v[pl.ds(16 * k, 16)] = jnp.ones((16,), jnp.float32)
    pltpu.sync_copy(zbuf, acc.at[pl.ds(s * RPT, RPT)])
    plsc.subcore_barrier()

    wbase = w * CPW
    pltpu.sync_copy(ep_hbm.at[wbase], ib0)

    def body(t, carry):
        a = wbase + 2 * t
        pltpu.make_async_copy(ep_hbm.at[a + 1], ib1, sem1).start()
        pltpu.sync_copy(ones_v, acc.at[ib0.at[1]], add=True)
        pltpu.make_async_copy(ep_hbm.at[a + 1], ib1, sem1).wait()
        pltpu.make_async_copy(ep_hbm.at[a + 2], ib0, sem0).start()
        pltpu.sync_copy(ones_v, acc.at[ib1.at[1]], add=True)
        pltpu.make_async_copy(ep_hbm.at[a + 2], ib0, sem0).wait()
        return carry

    lax.fori_loop(0, CPW // 2, body, 0)
    plsc.subcore_barrier()
    pltpu.sync_copy(acc.at[pl.ds(s * RPT, RPT)],
                    out_hbm.at[c].at[pl.ds(s * RPT, RPT)])


@functools.partial(jax.jit)
def _deg_call(ep):
    return pl.kernel(
        _deg_body,
        out_type=jax.ShapeDtypeStruct((NC, ROWS), jnp.float32),
        mesh=plsc.VectorSubcoreMesh(**_MESH),
        scratch_types=[
            pltpu.VMEM((2, CHUNK), jnp.int32),
            pltpu.VMEM((2, CHUNK), jnp.int32),
            pltpu.VMEM((CHUNK,), jnp.float32),
            pltpu.VMEM((RPT,), jnp.float32),
            pltpu.VMEM_SHARED((ROWS,), jnp.float32),
            pltpu.SemaphoreType.DMA,
            pltpu.SemaphoreType.DMA,
        ],
    )(ep)


def _scatter_body(y_hbm, ep_hbm, out_hbm, ib0, ib1, rb0, rb1, zbuf, acc,
                  sem0, sem1):
    c = lax.axis_index("c")
    s = lax.axis_index("s")
    w = c * NS + s

    for r in range(16):
        for k in range(H // 16):
            zbuf[r, pl.ds(16 * k, 16)] = jnp.zeros((16,), jnp.float32)

    def zloop(j, carry):
        pltpu.sync_copy(zbuf, acc.at[pl.ds(s * RPT + 16 * j, 16)])
        return carry

    lax.fori_loop(0, RPT // 16, zloop, 0)
    plsc.subcore_barrier()

    wbase = w * CPW
    pltpu.sync_copy(ep_hbm.at[wbase], ib0)
    pltpu.make_async_copy(y_hbm.at[ib0.at[0]], rb0, sem0).start()

    def body(t, carry):
        a = wbase + 2 * t
        pltpu.sync_copy(ep_hbm.at[a + 1], ib1)
        pltpu.make_async_copy(y_hbm.at[ib1.at[0]], rb1, sem1).start()
        pltpu.make_async_copy(y_hbm.at[ib0.at[0]], rb0, sem0).wait()
        pltpu.sync_copy(rb0, acc.at[ib0.at[1]], add=True)
        pltpu.sync_copy(ep_hbm.at[a + 2], ib0)
        pltpu.make_async_copy(y_hbm.at[ib0.at[0]], rb0, sem0).start()
        pltpu.make_async_copy(y_hbm.at[ib1.at[0]], rb1, sem1).wait()
        pltpu.sync_copy(rb1, acc.at[ib1.at[1]], add=True)
        return carry

    lax.fori_loop(0, CPW // 2, body, 0)
    pltpu.make_async_copy(y_hbm.at[ib0.at[0]], rb0, sem0).wait()
    plsc.subcore_barrier()
    pltpu.sync_copy(acc.at[pl.ds(s * RPT, RPT)],
                    out_hbm.at[c].at[pl.ds(s * RPT, RPT)])


@functools.partial(jax.jit)
def _scatter_call(y, ep):
    return pl.kernel(
        _scatter_body,
        out_type=jax.ShapeDtypeStruct((NC, ROWS, H), jnp.float32),
        mesh=plsc.VectorSubcoreMesh(**_MESH),
        scratch_types=[
            pltpu.VMEM((2, CHUNK), jnp.int32),
            pltpu.VMEM((2, CHUNK), jnp.int32),
            pltpu.VMEM((CHUNK, H), jnp.float32),
            pltpu.VMEM((CHUNK, H), jnp.float32),
            pltpu.VMEM((16, H), jnp.float32),
            pltpu.VMEM_SHARED((ROWS, H), jnp.float32),
            pltpu.SemaphoreType.DMA,
            pltpu.SemaphoreType.DMA,
        ],
        compiler_params=pltpu.CompilerParams(use_tc_tiling_on_sc=False),
    )(y, ep)


RB = 2000
NBLK = N // RB


def _t1_body(x_ref, w1_ref, degt_ref, y_ref, z_ref):
    deg = degt_ref[:, 0:1] + degt_ref[:, 1:2] + 1.0
    dis = lax.rsqrt(deg)
    xw = jnp.dot(x_ref[...], w1_ref[...], preferred_element_type=jnp.float32)
    y_ref[...] = xw * dis
    z_ref[...] = xw / deg


def _t1(x, W1, degt):
    return pl.pallas_call(
        _t1_body,
        grid=(NBLK,),
        in_specs=[
            pl.BlockSpec((RB, DIN), lambda i: (i, 0)),
            pl.BlockSpec((DIN, H), lambda i: (0, 0)),
            pl.BlockSpec((RB, 2), lambda i: (i, 0)),
        ],
        out_specs=[
            pl.BlockSpec((RB, H), lambda i: (i, 0)),
            pl.BlockSpec((RB, H), lambda i: (i, 0)),
        ],
        out_shape=[
            jax.ShapeDtypeStruct((N, H), jnp.float32),
            jax.ShapeDtypeStruct((N, H), jnp.float32),
        ],
    )(x, W1, degt)


def _t2_body(acc_ref, z1_ref, degt_ref, w2_ref, b1_ref, y_ref, z_ref):
    deg = degt_ref[:, 0:1] + degt_ref[:, 1:2] + 1.0
    dis = lax.rsqrt(deg)
    a = acc_ref[0] + acc_ref[1]
    h1 = jnp.maximum(a * dis + z1_ref[...] + b1_ref[...], 0.0)
    xw = jnp.dot(h1, w2_ref[...], preferred_element_type=jnp.float32)
    y_ref[...] = xw * dis
    z_ref[...] = xw / deg


def _t2(acc1, z1, degt, W2, b1r):
    return pl.pallas_call(
        _t2_body,
        grid=(NBLK,),
        in_specs=[
            pl.BlockSpec((NC, RB, H), lambda i: (0, i, 0)),
            pl.BlockSpec((RB, H), lambda i: (i, 0)),
            pl.BlockSpec((RB, 2), lambda i: (i, 0)),
            pl.BlockSpec((H, H), lambda i: (0, 0)),
            pl.BlockSpec((1, H), lambda i: (0, 0)),
        ],
        out_specs=[
            pl.BlockSpec((RB, H), lambda i: (i, 0)),
            pl.BlockSpec((RB, H), lambda i: (i, 0)),
        ],
        out_shape=[
            jax.ShapeDtypeStruct((N, H), jnp.float32),
            jax.ShapeDtypeStruct((N, H), jnp.float32),
        ],
    )(acc1, z1, degt, W2, b1r)


def _t3_body(acc_ref, z2_ref, degt_ref, bcol_ref, b2_ref, wl1_ref, bl1_ref,
             wl2_ref, bl2_ref, out_ref, gsum, cnt):
    i = pl.program_id(0)

    @pl.when(i == 0)
    def _():
        gsum[...] = jnp.zeros((G, H), jnp.float32)
        cnt[...] = jnp.zeros((G, 1), jnp.float32)

    deg = degt_ref[:, 0:1] + degt_ref[:, 1:2] + 1.0
    dis = lax.rsqrt(deg)
    h2 = (acc_ref[0] + acc_ref[1]) * dis + z2_ref[...] + b2_ref[...]
    pt = (bcol_ref[...] == lax.broadcasted_iota(jnp.int32, (1, G), 1))
    pt = pt.astype(jnp.float32)
    dn = (((0,), (0,)), ((), ()))
    gsum[...] += lax.dot_general(pt, h2, dn,
                                 preferred_element_type=jnp.float32)
    cnt[...] += lax.dot_general(pt, jnp.ones((RB, 1), jnp.float32), dn,
                                preferred_element_type=jnp.float32)

    @pl.when(i == NBLK - 1)
    def _():
        g = gsum[...] / jnp.maximum(cnt[...], 1.0)
        g = jnp.dot(g, wl1_ref[...],
                    preferred_element_type=jnp.float32) + bl1_ref[...]
        g = jnp.dot(g, wl2_ref[...],
                    preferred_element_type=jnp.float32) + bl2_ref[...]
        m = jnp.max(g, axis=0, keepdims=True)
        e = jnp.exp(g - m)
        out_ref[...] = e / jnp.sum(e, axis=0, keepdims=True)


def _t3(acc2, z2, degt, bcol, b2r, Wl1, bl1r, Wl2, bl2r):
    return pl.pallas_call(
        _t3_body,
        grid=(NBLK,),
        in_specs=[
            pl.BlockSpec((NC, RB, H), lambda i: (0, i, 0)),
            pl.BlockSpec((RB, H), lambda i: (i, 0)),
            pl.BlockSpec((RB, 2), lambda i: (i, 0)),
            pl.BlockSpec((RB, 1), lambda i: (i, 0)),
            pl.BlockSpec((1, H), lambda i: (0, 0)),
            pl.BlockSpec((H, 32), lambda i: (0, 0)),
            pl.BlockSpec((1, 32), lambda i: (0, 0)),
            pl.BlockSpec((32, 2), lambda i: (0, 0)),
            pl.BlockSpec((1, 2), lambda i: (0, 0)),
        ],
        out_specs=pl.BlockSpec((G, 2), lambda i: (0, 0)),
        out_shape=jax.ShapeDtypeStruct((G, 2), jnp.float32),
        scratch_shapes=[
            pltpu.VMEM((G, H), jnp.float32),
            pltpu.VMEM((G, 1), jnp.float32),
        ],
    )(acc2, z2, degt, bcol, b2r, Wl1, bl1r, Wl2, bl2r)


def kernel(x, edge_index, batch, W1, b1, W2, b2, Wl1, bl1, Wl2, bl2):
    src = edge_index[0].astype(jnp.int32)
    dst = edge_index[1].astype(jnp.int32)
    pad = EPAD + CHUNK - E
    src_p = jnp.concatenate([src, jnp.zeros((pad,), jnp.int32)])
    dst_p = jnp.concatenate([dst, jnp.full((pad,), N, jnp.int32)])
    ep = jnp.stack([src_p.reshape(NCH + 1, CHUNK),
                    dst_p.reshape(NCH + 1, CHUNK)], axis=1)

    degp = _deg_call(ep)
    degt = degp.T[:N]

    y1, z1 = _t1(x, W1, degt)
    acc1 = _scatter_call(y1, ep)
    y2, z2 = _t2(acc1, z1, degt, W2, b1.reshape(1, H))
    acc2 = _scatter_call(y2, ep)
    return _t3(acc2, z2, degt, batch.astype(jnp.int32).reshape(N, 1),
               b2.reshape(1, H), Wl1, bl1.reshape(1, 32),
               Wl2, bl2.reshape(1, 2))


# speedup vs baseline: 27.4249x; 1.7387x over previous
import functools

import jax
import jax.numpy as jnp
from jax import lax
from jax.experimental import pallas as pl
from jax.experimental.pallas import tpu as pltpu
from jax.experimental.pallas import tpu_sc as plsc

N = 10000
E = 320000
DIN = 128
H = 64
G = 128

NC = 2
NS = 16
NW = NC * NS
CHUNK = 128
CPW = 80
NCH = NW * CPW
EPAD = NCH * CHUNK
ROWS = 10240
RPT = ROWS // NS

_MESH = dict(core_axis_name="c", subcore_axis_name="s",
             num_cores=NC, num_subcores=NS)


def _zero_f32_ref(ref, n):
    for k in range(n // 16):
        ref[pl.ds(16 * k, 16)] = jnp.zeros((16,), jnp.float32)


def _deg_body(ep_hbm, out_hbm, ib0, ib1, ones_v, zbuf, acc, sem0, sem1):
    c = lax.axis_index("c")
    s = lax.axis_index("s")
    w = c * NS + s

    _zero_f32_ref(zbuf, RPT)
    for k in range(CHUNK // 16):
        ones_v[pl.ds(16 * k, 16)] = jnp.ones((16,), jnp.float32)
    pltpu.sync_copy(zbuf, acc.at[pl.ds(s * RPT, RPT)])
    plsc.subcore_barrier()

    wbase = w * CPW
    pltpu.sync_copy(ep_hbm.at[wbase], ib0)

    def body(t, carry):
        a = wbase + 2 * t
        pltpu.make_async_copy(ep_hbm.at[a + 1], ib1, sem1).start()
        pltpu.sync_copy(ones_v, acc.at[ib0.at[1]], add=True)
        pltpu.make_async_copy(ep_hbm.at[a + 1], ib1, sem1).wait()
        pltpu.make_async_copy(ep_hbm.at[a + 2], ib0, sem0).start()
        pltpu.sync_copy(ones_v, acc.at[ib1.at[1]], add=True)
        pltpu.make_async_copy(ep_hbm.at[a + 2], ib0, sem0).wait()
        return carry

    lax.fori_loop(0, CPW // 2, body, 0)
    plsc.subcore_barrier()
    pltpu.sync_copy(acc.at[pl.ds(s * RPT, RPT)],
                    out_hbm.at[c].at[pl.ds(s * RPT, RPT)])


@functools.partial(jax.jit)
def _deg_call(ep):
    return pl.kernel(
        _deg_body,
        out_type=jax.ShapeDtypeStruct((NC, ROWS), jnp.float32),
        mesh=plsc.VectorSubcoreMesh(**_MESH),
        scratch_types=[
            pltpu.VMEM((2, CHUNK), jnp.int32),
            pltpu.VMEM((2, CHUNK), jnp.int32),
            pltpu.VMEM((CHUNK,), jnp.float32),
            pltpu.VMEM((RPT,), jnp.float32),
            pltpu.VMEM_SHARED((ROWS,), jnp.float32),
            pltpu.SemaphoreType.DMA,
            pltpu.SemaphoreType.DMA,
        ],
    )(ep)


def _scatter_body(y_hbm, ep_hbm, out_hbm, ib0, ib1, rb0, rb1, zbuf, acc,
                  ytab, sem0, sem1, semy):
    c = lax.axis_index("c")
    s = lax.axis_index("s")
    w = c * NS + s

    pltpu.make_async_copy(y_hbm.at[pl.ds(s * (N // NS), N // NS)],
                          ytab.at[pl.ds(s * (N // NS), N // NS)],
                          semy).start()

    for r in range(16):
        for k in range(H // 16):
            zbuf[r, pl.ds(16 * k, 16)] = jnp.zeros((16,), jnp.float32)

    def zloop(j, carry):
        pltpu.sync_copy(zbuf, acc.at[pl.ds(s * RPT + 16 * j, 16)])
        return carry

    lax.fori_loop(0, RPT // 16, zloop, 0)
    pltpu.make_async_copy(y_hbm.at[pl.ds(s * (N // NS), N // NS)],
                          ytab.at[pl.ds(s * (N // NS), N // NS)],
                          semy).wait()
    plsc.subcore_barrier()

    wbase = w * CPW
    pltpu.sync_copy(ep_hbm.at[wbase], ib0)
    pltpu.make_async_copy(ytab.at[ib0.at[0]], rb0, sem0).start()

    def body(t, carry):
        a = wbase + 2 * t
        pltpu.sync_copy(ep_hbm.at[a + 1], ib1)
        pltpu.make_async_copy(ytab.at[ib1.at[0]], rb1, sem1).start()
        pltpu.make_async_copy(ytab.at[ib0.at[0]], rb0, sem0).wait()
        pltpu.sync_copy(rb0, acc.at[ib0.at[1]], add=True)
        pltpu.sync_copy(ep_hbm.at[a + 2], ib0)
        pltpu.make_async_copy(ytab.at[ib0.at[0]], rb0, sem0).start()
        pltpu.make_async_copy(ytab.at[ib1.at[0]], rb1, sem1).wait()
        pltpu.sync_copy(rb1, acc.at[ib1.at[1]], add=True)
        return carry

    lax.fori_loop(0, CPW // 2, body, 0)
    pltpu.make_async_copy(ytab.at[ib0.at[0]], rb0, sem0).wait()
    plsc.subcore_barrier()
    pltpu.sync_copy(acc.at[pl.ds(s * RPT, RPT)],
                    out_hbm.at[c].at[pl.ds(s * RPT, RPT)])


@functools.partial(jax.jit)
def _scatter_call(y, ep):
    return pl.kernel(
        _scatter_body,
        out_type=jax.ShapeDtypeStruct((NC, ROWS, H), jnp.float32),
        mesh=plsc.VectorSubcoreMesh(**_MESH),
        scratch_types=[
            pltpu.VMEM((2, CHUNK), jnp.int32),
            pltpu.VMEM((2, CHUNK), jnp.int32),
            pltpu.VMEM((CHUNK, H), jnp.float32),
            pltpu.VMEM((CHUNK, H), jnp.float32),
            pltpu.VMEM((16, H), jnp.float32),
            pltpu.VMEM_SHARED((ROWS, H), jnp.float32),
            pltpu.VMEM_SHARED((N, H), jnp.float32),
            pltpu.SemaphoreType.DMA,
            pltpu.SemaphoreType.DMA,
            pltpu.SemaphoreType.DMA,
        ],
        compiler_params=pltpu.CompilerParams(use_tc_tiling_on_sc=False),
    )(y, ep)


RB = 2000
NBLK = N // RB


def _t1_body(x_ref, w1_ref, degt_ref, y_ref, z_ref):
    deg = degt_ref[:, 0:1] + degt_ref[:, 1:2] + 1.0
    dis = lax.rsqrt(deg)
    xw = jnp.dot(x_ref[...], w1_ref[...], preferred_element_type=jnp.float32)
    y_ref[...] = xw * dis
    z_ref[...] = xw / deg


def _t1(x, W1, degt):
    return pl.pallas_call(
        _t1_body,
        grid=(NBLK,),
        in_specs=[
            pl.BlockSpec((RB, DIN), lambda i: (i, 0)),
            pl.BlockSpec((DIN, H), lambda i: (0, 0)),
            pl.BlockSpec((RB, 2), lambda i: (i, 0)),
        ],
        out_specs=[
            pl.BlockSpec((RB, H), lambda i: (i, 0)),
            pl.BlockSpec((RB, H), lambda i: (i, 0)),
        ],
        out_shape=[
            jax.ShapeDtypeStruct((N, H), jnp.float32),
            jax.ShapeDtypeStruct((N, H), jnp.float32),
        ],
    )(x, W1, degt)


def _t2_body(acc_ref, z1_ref, degt_ref, w2_ref, b1_ref, y_ref, z_ref):
    deg = degt_ref[:, 0:1] + degt_ref[:, 1:2] + 1.0
    dis = lax.rsqrt(deg)
    a = acc_ref[0] + acc_ref[1]
    h1 = jnp.maximum(a * dis + z1_ref[...] + b1_ref[...], 0.0)
    xw = jnp.dot(h1, w2_ref[...], preferred_element_type=jnp.float32)
    y_ref[...] = xw * dis
    z_ref[...] = xw / deg


def _t2(acc1, z1, degt, W2, b1r):
    return pl.pallas_call(
        _t2_body,
        grid=(NBLK,),
        in_specs=[
            pl.BlockSpec((NC, RB, H), lambda i: (0, i, 0)),
            pl.BlockSpec((RB, H), lambda i: (i, 0)),
            pl.BlockSpec((RB, 2), lambda i: (i, 0)),
            pl.BlockSpec((H, H), lambda i: (0, 0)),
            pl.BlockSpec((1, H), lambda i: (0, 0)),
        ],
        out_specs=[
            pl.BlockSpec((RB, H), lambda i: (i, 0)),
            pl.BlockSpec((RB, H), lambda i: (i, 0)),
        ],
        out_shape=[
            jax.ShapeDtypeStruct((N, H), jnp.float32),
            jax.ShapeDtypeStruct((N, H), jnp.float32),
        ],
    )(acc1, z1, degt, W2, b1r)


def _t3_body(acc_ref, z2_ref, degt_ref, bcol_ref, b2_ref, wl1_ref, bl1_ref,
             wl2_ref, bl2_ref, out_ref, gsum, cnt):
    i = pl.program_id(0)

    @pl.when(i == 0)
    def _():
        gsum[...] = jnp.zeros((G, H), jnp.float32)
        cnt[...] = jnp.zeros((G, 1), jnp.float32)

    deg = degt_ref[:, 0:1] + degt_ref[:, 1:2] + 1.0
    dis = lax.rsqrt(deg)
    h2 = (acc_ref[0] + acc_ref[1]) * dis + z2_ref[...] + b2_ref[...]
    pt = (bcol_ref[...] == lax.broadcasted_iota(jnp.int32, (1, G), 1))
    pt = pt.astype(jnp.float32)
    dn = (((0,), (0,)), ((), ()))
    gsum[...] += lax.dot_general(pt, h2, dn,
                                 preferred_element_type=jnp.float32)
    cnt[...] += lax.dot_general(pt, jnp.ones((RB, 1), jnp.float32), dn,
                                preferred_element_type=jnp.float32)

    @pl.when(i == NBLK - 1)
    def _():
        g = gsum[...] / jnp.maximum(cnt[...], 1.0)
        g = jnp.dot(g, wl1_ref[...],
                    preferred_element_type=jnp.float32) + bl1_ref[...]
        g = jnp.dot(g, wl2_ref[...],
                    preferred_element_type=jnp.float32) + bl2_ref[...]
        m = jnp.max(g, axis=0, keepdims=True)
        e = jnp.exp(g - m)
        out_ref[...] = e / jnp.sum(e, axis=0, keepdims=True)


def _t3(acc2, z2, degt, bcol, b2r, Wl1, bl1r, Wl2, bl2r):
    return pl.pallas_call(
        _t3_body,
        grid=(NBLK,),
        in_specs=[
            pl.BlockSpec((NC, RB, H), lambda i: (0, i, 0)),
            pl.BlockSpec((RB, H), lambda i: (i, 0)),
            pl.BlockSpec((RB, 2), lambda i: (i, 0)),
            pl.BlockSpec((RB, 1), lambda i: (i, 0)),
            pl.BlockSpec((1, H), lambda i: (0, 0)),
            pl.BlockSpec((H, 32), lambda i: (0, 0)),
            pl.BlockSpec((1, 32), lambda i: (0, 0)),
            pl.BlockSpec((32, 2), lambda i: (0, 0)),
            pl.BlockSpec((1, 2), lambda i: (0, 0)),
        ],
        out_specs=pl.BlockSpec((G, 2), lambda i: (0, 0)),
        out_shape=jax.ShapeDtypeStruct((G, 2), jnp.float32),
        scratch_shapes=[
            pltpu.VMEM((G, H), jnp.float32),
            pltpu.VMEM((G, 1), jnp.float32),
        ],
    )(acc2, z2, degt, bcol, b2r, Wl1, bl1r, Wl2, bl2r)


def kernel(x, edge_index, batch, W1, b1, W2, b2, Wl1, bl1, Wl2, bl2):
    src = edge_index[0].astype(jnp.int32)
    dst = edge_index[1].astype(jnp.int32)
    pad = EPAD + CHUNK - E
    src_p = jnp.concatenate([src, jnp.zeros((pad,), jnp.int32)])
    dst_p = jnp.concatenate([dst, jnp.full((pad,), N, jnp.int32)])
    ep = jnp.stack([src_p.reshape(NCH + 1, CHUNK),
                    dst_p.reshape(NCH + 1, CHUNK)], axis=1)

    degp = _deg_call(ep)
    degt = degp.T[:N]

    y1, z1 = _t1(x, W1, degt)
    acc1 = _scatter_call(y1, ep)
    y2, z2 = _t2(acc1, z1, degt, W2, b1.reshape(1, H))
    acc2 = _scatter_call(y2, ep)
    return _t3(acc2, z2, degt, batch.astype(jnp.int32).reshape(N, 1),
               b2.reshape(1, H), Wl1, bl1.reshape(1, 32),
               Wl2, bl2.reshape(1, 2))


# speedup vs baseline: 32.6705x; 1.1913x over previous
import functools

import jax
import jax.numpy as jnp
from jax import lax
from jax.experimental import pallas as pl
from jax.experimental.pallas import tpu as pltpu
from jax.experimental.pallas import tpu_sc as plsc

N = 10000
E = 320000
DIN = 128
H = 64
G = 128

NC = 2
NS = 16
NW = NC * NS
CHUNK = 128
CPW = 80
NCH = NW * CPW
EPAD = NCH * CHUNK
ROWS = 10240
RPT = ROWS // NS

_MESH = dict(core_axis_name="c", subcore_axis_name="s",
             num_cores=NC, num_subcores=NS)


def _zero_f32_ref(ref, n):
    for k in range(n // 16):
        ref[pl.ds(16 * k, 16)] = jnp.zeros((16,), jnp.float32)


def _deg_body(ep_hbm, out_hbm, ib0, ib1, ones_v, zbuf, acc, sem0, sem1):
    c = lax.axis_index("c")
    s = lax.axis_index("s")
    w = c * NS + s

    _zero_f32_ref(zbuf, RPT)
    for k in range(CHUNK // 16):
        ones_v[pl.ds(16 * k, 16)] = jnp.ones((16,), jnp.float32)
    pltpu.sync_copy(zbuf, acc.at[pl.ds(s * RPT, RPT)])
    plsc.subcore_barrier()

    wbase = w * CPW
    pltpu.sync_copy(ep_hbm.at[wbase], ib0)

    def body(t, carry):
        a = wbase + 2 * t
        pltpu.make_async_copy(ep_hbm.at[a + 1], ib1, sem1).start()
        pltpu.sync_copy(ones_v, acc.at[ib0.at[1]], add=True)
        pltpu.make_async_copy(ep_hbm.at[a + 1], ib1, sem1).wait()
        pltpu.make_async_copy(ep_hbm.at[a + 2], ib0, sem0).start()
        pltpu.sync_copy(ones_v, acc.at[ib1.at[1]], add=True)
        pltpu.make_async_copy(ep_hbm.at[a + 2], ib0, sem0).wait()
        return carry

    lax.fori_loop(0, CPW // 2, body, 0)
    plsc.subcore_barrier()
    pltpu.sync_copy(acc.at[pl.ds(s * RPT, RPT)],
                    out_hbm.at[c].at[pl.ds(s * RPT, RPT)])


@functools.partial(jax.jit)
def _deg_call(ep):
    return pl.kernel(
        _deg_body,
        out_type=jax.ShapeDtypeStruct((NC, ROWS), jnp.float32),
        mesh=plsc.VectorSubcoreMesh(**_MESH),
        scratch_types=[
            pltpu.VMEM((2, CHUNK), jnp.int32),
            pltpu.VMEM((2, CHUNK), jnp.int32),
            pltpu.VMEM((CHUNK,), jnp.float32),
            pltpu.VMEM((RPT,), jnp.float32),
            pltpu.VMEM_SHARED((ROWS,), jnp.float32),
            pltpu.SemaphoreType.DMA,
            pltpu.SemaphoreType.DMA,
        ],
    )(ep)


def _scatter_body(y_hbm, ep_hbm, out_hbm, ib0, ib1, rb0, rb1, zbuf, acc,
                  ytab, sem0, sem1, semi0, semi1, semy):
    c = lax.axis_index("c")
    s = lax.axis_index("s")
    w = c * NS + s

    pltpu.make_async_copy(y_hbm.at[pl.ds(s * (N // NS), N // NS)],
                          ytab.at[pl.ds(s * (N // NS), N // NS)],
                          semy).start()

    for r in range(16):
        for k in range(H // 16):
            zbuf[r, pl.ds(16 * k, 16)] = jnp.zeros((16,), jnp.float32)

    def zloop(j, carry):
        pltpu.sync_copy(zbuf, acc.at[pl.ds(s * RPT + 16 * j, 16)])
        return carry

    lax.fori_loop(0, RPT // 16, zloop, 0)
    pltpu.make_async_copy(y_hbm.at[pl.ds(s * (N // NS), N // NS)],
                          ytab.at[pl.ds(s * (N // NS), N // NS)],
                          semy).wait()
    plsc.subcore_barrier()

    wbase = w * CPW
    pltpu.sync_copy(ep_hbm.at[pl.ds(wbase, 2)], ib0)
    pltpu.make_async_copy(ep_hbm.at[pl.ds(wbase + 2, 2)], ib1, semi1).start()
    pltpu.make_async_copy(ytab.at[ib0.at[0].at[0]], rb0, sem0).start()

    def body(u, carry):
        a = wbase + 4 * u
        pltpu.make_async_copy(ytab.at[ib0.at[1].at[0]], rb1, sem1).start()
        pltpu.make_async_copy(ytab.at[ib0.at[0].at[0]], rb0, sem0).wait()
        pltpu.sync_copy(rb0, acc.at[ib0.at[0].at[1]], add=True)
        pltpu.make_async_copy(ep_hbm.at[pl.ds(a + 2, 2)], ib1, semi1).wait()
        pltpu.make_async_copy(ytab.at[ib1.at[0].at[0]], rb0, sem0).start()
        pltpu.make_async_copy(ytab.at[ib0.at[1].at[0]], rb1, sem1).wait()
        pltpu.sync_copy(rb1, acc.at[ib0.at[1].at[1]], add=True)
        pltpu.make_async_copy(ep_hbm.at[pl.ds(a + 4, 2)], ib0, semi0).start()
        pltpu.make_async_copy(ytab.at[ib1.at[1].at[0]], rb1, sem1).start()
        pltpu.make_async_copy(ytab.at[ib1.at[0].at[0]], rb0, sem0).wait()
        pltpu.sync_copy(rb0, acc.at[ib1.at[0].at[1]], add=True)
        pltpu.make_async_copy(ep_hbm.at[pl.ds(a + 4, 2)], ib0, semi0).wait()
        pltpu.make_async_copy(ytab.at[ib0.at[0].at[0]], rb0, sem0).start()
        pltpu.make_async_copy(ytab.at[ib1.at[1].at[0]], rb1, sem1).wait()
        pltpu.sync_copy(rb1, acc.at[ib1.at[1].at[1]], add=True)
        pltpu.make_async_copy(ep_hbm.at[pl.ds(a + 6, 2)], ib1, semi1).start()
        return carry

    lax.fori_loop(0, CPW // 4, body, 0)
    pltpu.make_async_copy(ytab.at[ib0.at[0].at[0]], rb0, sem0).wait()
    pltpu.make_async_copy(ep_hbm.at[pl.ds(wbase + CPW + 2, 2)], ib1,
                          semi1).wait()
    plsc.subcore_barrier()
    pltpu.sync_copy(acc.at[pl.ds(s * RPT, RPT)],
                    out_hbm.at[c].at[pl.ds(s * RPT, RPT)])


@functools.partial(jax.jit)
def _scatter_call(y, ep):
    return pl.kernel(
        _scatter_body,
        out_type=jax.ShapeDtypeStruct((NC, ROWS, H), jnp.float32),
        mesh=plsc.VectorSubcoreMesh(**_MESH),
        scratch_types=[
            pltpu.VMEM((2, 2, CHUNK), jnp.int32),
            pltpu.VMEM((2, 2, CHUNK), jnp.int32),
            pltpu.VMEM((CHUNK, H), jnp.float32),
            pltpu.VMEM((CHUNK, H), jnp.float32),
            pltpu.VMEM((16, H), jnp.float32),
            pltpu.VMEM_SHARED((ROWS, H), jnp.float32),
            pltpu.VMEM_SHARED((N, H), jnp.float32),
            pltpu.SemaphoreType.DMA,
            pltpu.SemaphoreType.DMA,
            pltpu.SemaphoreType.DMA,
            pltpu.SemaphoreType.DMA,
            pltpu.SemaphoreType.DMA,
        ],
        compiler_params=pltpu.CompilerParams(use_tc_tiling_on_sc=False),
    )(y, ep)


RB = 2000
NBLK = N // RB


def _t1_body(x_ref, w1_ref, degt_ref, y_ref, z_ref):
    deg = degt_ref[:, 0:1] + degt_ref[:, 1:2] + 1.0
    dis = lax.rsqrt(deg)
    xw = jnp.dot(x_ref[...], w1_ref[...], preferred_element_type=jnp.float32)
    y_ref[...] = xw * dis
    z_ref[...] = xw / deg


def _t1(x, W1, degt):
    return pl.pallas_call(
        _t1_body,
        grid=(NBLK,),
        in_specs=[
            pl.BlockSpec((RB, DIN), lambda i: (i, 0)),
            pl.BlockSpec((DIN, H), lambda i: (0, 0)),
            pl.BlockSpec((RB, 2), lambda i: (i, 0)),
        ],
        out_specs=[
            pl.BlockSpec((RB, H), lambda i: (i, 0)),
            pl.BlockSpec((RB, H), lambda i: (i, 0)),
        ],
        out_shape=[
            jax.ShapeDtypeStruct((N, H), jnp.float32),
            jax.ShapeDtypeStruct((N, H), jnp.float32),
        ],
    )(x, W1, degt)


def _t2_body(acc_ref, z1_ref, degt_ref, w2_ref, b1_ref, y_ref, z_ref):
    deg = degt_ref[:, 0:1] + degt_ref[:, 1:2] + 1.0
    dis = lax.rsqrt(deg)
    a = acc_ref[0] + acc_ref[1]
    h1 = jnp.maximum(a * dis + z1_ref[...] + b1_ref[...], 0.0)
    xw = jnp.dot(h1, w2_ref[...], preferred_element_type=jnp.float32)
    y_ref[...] = xw * dis
    z_ref[...] = xw / deg


def _t2(acc1, z1, degt, W2, b1r):
    return pl.pallas_call(
        _t2_body,
        grid=(NBLK,),
        in_specs=[
            pl.BlockSpec((NC, RB, H), lambda i: (0, i, 0)),
            pl.BlockSpec((RB, H), lambda i: (i, 0)),
            pl.BlockSpec((RB, 2), lambda i: (i, 0)),
            pl.BlockSpec((H, H), lambda i: (0, 0)),
            pl.BlockSpec((1, H), lambda i: (0, 0)),
        ],
        out_specs=[
            pl.BlockSpec((RB, H), lambda i: (i, 0)),
            pl.BlockSpec((RB, H), lambda i: (i, 0)),
        ],
        out_shape=[
            jax.ShapeDtypeStruct((N, H), jnp.float32),
            jax.ShapeDtypeStruct((N, H), jnp.float32),
        ],
    )(acc1, z1, degt, W2, b1r)


def _t3_body(acc_ref, z2_ref, degt_ref, bcol_ref, b2_ref, wl1_ref, bl1_ref,
             wl2_ref, bl2_ref, out_ref, gsum, cnt):
    i = pl.program_id(0)

    @pl.when(i == 0)
    def _():
        gsum[...] = jnp.zeros((G, H), jnp.float32)
        cnt[...] = jnp.zeros((G, 1), jnp.float32)

    deg = degt_ref[:, 0:1] + degt_ref[:, 1:2] + 1.0
    dis = lax.rsqrt(deg)
    h2 = (acc_ref[0] + acc_ref[1]) * dis + z2_ref[...] + b2_ref[...]
    pt = (bcol_ref[...] == lax.broadcasted_iota(jnp.int32, (1, G), 1))
    pt = pt.astype(jnp.float32)
    dn = (((0,), (0,)), ((), ()))
    gsum[...] += lax.dot_general(pt, h2, dn,
                                 preferred_element_type=jnp.float32)
    cnt[...] += lax.dot_general(pt, jnp.ones((RB, 1), jnp.float32), dn,
                                preferred_element_type=jnp.float32)

    @pl.when(i == NBLK - 1)
    def _():
        g = gsum[...] / jnp.maximum(cnt[...], 1.0)
        g = jnp.dot(g, wl1_ref[...],
                    preferred_element_type=jnp.float32) + bl1_ref[...]
        g = jnp.dot(g, wl2_ref[...],
                    preferred_element_type=jnp.float32) + bl2_ref[...]
        m = jnp.max(g, axis=0, keepdims=True)
        e = jnp.exp(g - m)
        out_ref[...] = e / jnp.sum(e, axis=0, keepdims=True)


def _t3(acc2, z2, degt, bcol, b2r, Wl1, bl1r, Wl2, bl2r):
    return pl.pallas_call(
        _t3_body,
        grid=(NBLK,),
        in_specs=[
            pl.BlockSpec((NC, RB, H), lambda i: (0, i, 0)),
            pl.BlockSpec((RB, H), lambda i: (i, 0)),
            pl.BlockSpec((RB, 2), lambda i: (i, 0)),
            pl.BlockSpec((RB, 1), lambda i: (i, 0)),
            pl.BlockSpec((1, H), lambda i: (0, 0)),
            pl.BlockSpec((H, 32), lambda i: (0, 0)),
            pl.BlockSpec((1, 32), lambda i: (0, 0)),
            pl.BlockSpec((32, 2), lambda i: (0, 0)),
            pl.BlockSpec((1, 2), lambda i: (0, 0)),
        ],
        out_specs=pl.BlockSpec((G, 2), lambda i: (0, 0)),
        out_shape=jax.ShapeDtypeStruct((G, 2), jnp.float32),
        scratch_shapes=[
            pltpu.VMEM((G, H), jnp.float32),
            pltpu.VMEM((G, 1), jnp.float32),
        ],
    )(acc2, z2, degt, bcol, b2r, Wl1, bl1r, Wl2, bl2r)


def kernel(x, edge_index, batch, W1, b1, W2, b2, Wl1, bl1, Wl2, bl2):
    src = edge_index[0].astype(jnp.int32)
    dst = edge_index[1].astype(jnp.int32)
    pad = EPAD + 4 * CHUNK - E
    src_p = jnp.concatenate([src, jnp.zeros((pad,), jnp.int32)])
    dst_p = jnp.concatenate([dst, jnp.full((pad,), N, jnp.int32)])
    ep = jnp.stack([src_p.reshape(NCH + 4, CHUNK),
                    dst_p.reshape(NCH + 4, CHUNK)], axis=1)

    degp = _deg_call(ep)
    degt = degp.T[:N]

    y1, z1 = _t1(x, W1, degt)
    acc1 = _scatter_call(y1, ep)
    y2, z2 = _t2(acc1, z1, degt, W2, b1.reshape(1, H))
    acc2 = _scatter_call(y2, ep)
    return _t3(acc2, z2, degt, batch.astype(jnp.int32).reshape(N, 1),
               b2.reshape(1, H), Wl1, bl1.reshape(1, 32),
               Wl2, bl2.reshape(1, 2))


# speedup vs baseline: 34.2136x; 1.0472x over previous
import functools

import jax
import jax.numpy as jnp
from jax import lax
from jax.experimental import pallas as pl
from jax.experimental.pallas import tpu as pltpu
from jax.experimental.pallas import tpu_sc as plsc

N = 10000
E = 320000
DIN = 128
H = 64
G = 128

NC = 2
NS = 16
NW = NC * NS
CHUNK = 128
CPW = 80
NCH = NW * CPW
EPAD = NCH * CHUNK
ROWS = 10240
RPT = ROWS // NS

_MESH = dict(core_axis_name="c", subcore_axis_name="s",
             num_cores=NC, num_subcores=NS)


def _zero_f32_ref(ref, n):
    for k in range(n // 16):
        ref[pl.ds(16 * k, 16)] = jnp.zeros((16,), jnp.float32)


def _deg_body(ep_hbm, out_hbm, ib0, ib1, dhist, tbuf, tmp, slab,
              semi0, semi1):
    c = lax.axis_index("c")
    s = lax.axis_index("s")
    w = c * NS + s

    def zero_hist(j, carry):
        dhist[pl.ds(16 * j, 16)] = jnp.zeros((16,), jnp.float32)
        return carry

    lax.fori_loop(0, ROWS // 16, zero_hist, 0)

    ones = jnp.ones((16,), jnp.float32)
    wbase = w * CPW
    pltpu.sync_copy(ep_hbm.at[pl.ds(wbase, 2)], ib0)
    pltpu.make_async_copy(ep_hbm.at[pl.ds(wbase + 2, 2)], ib1, semi1).start()

    def hist_pair(ib):
        for p in range(2):
            for k in range(CHUNK // 16):
                idx = ib[p, 1, pl.ds(16 * k, 16)]
                plsc.addupdate_scatter(dhist, [idx], ones)

    def body(u, carry):
        a = wbase + 4 * u
        hist_pair(ib0)
        pltpu.make_async_copy(ep_hbm.at[pl.ds(a + 2, 2)], ib1, semi1).wait()
        pltpu.make_async_copy(ep_hbm.at[pl.ds(a + 4, 2)], ib0, semi0).start()
        hist_pair(ib1)
        pltpu.make_async_copy(ep_hbm.at[pl.ds(a + 4, 2)], ib0, semi0).wait()
        pltpu.make_async_copy(ep_hbm.at[pl.ds(a + 6, 2)], ib1, semi1).start()
        return carry

    lax.fori_loop(0, CPW // 4, body, 0)
    pltpu.make_async_copy(ep_hbm.at[pl.ds(wbase + CPW + 2, 2)], ib1,
                          semi1).wait()

    pltpu.sync_copy(dhist, slab.at[s])
    plsc.subcore_barrier()
    pltpu.sync_copy(slab.at[:, pl.ds(s * RPT, RPT)], tbuf)

    def red(j, carry):
        o = 16 * j
        v = tbuf[0, pl.ds(o, 16)]
        for k in range(1, NS):
            v = v + tbuf[k, pl.ds(o, 16)]
        tmp[pl.ds(o, 16)] = v
        return carry

    lax.fori_loop(0, RPT // 16, red, 0)
    pltpu.sync_copy(tmp, out_hbm.at[c].at[pl.ds(s * RPT, RPT)])


@functools.partial(jax.jit)
def _deg_call(ep):
    return pl.kernel(
        _deg_body,
        out_type=jax.ShapeDtypeStruct((NC, ROWS), jnp.float32),
        mesh=plsc.VectorSubcoreMesh(**_MESH),
        scratch_types=[
            pltpu.VMEM((2, 2, CHUNK), jnp.int32),
            pltpu.VMEM((2, 2, CHUNK), jnp.int32),
            pltpu.VMEM((ROWS,), jnp.float32),
            pltpu.VMEM((NS, RPT), jnp.float32),
            pltpu.VMEM((RPT,), jnp.float32),
            pltpu.VMEM_SHARED((NS, ROWS), jnp.float32),
            pltpu.SemaphoreType.DMA,
            pltpu.SemaphoreType.DMA,
        ],
        compiler_params=pltpu.CompilerParams(needs_layout_passes=False),
    )(ep)


def _scatter_body(y_hbm, ep_hbm, out_hbm, ib0, ib1, rb0, rb1, zbuf, acc,
                  ytab, sem0, sem1, semi0, semi1, semy):
    c = lax.axis_index("c")
    s = lax.axis_index("s")
    w = c * NS + s

    pltpu.make_async_copy(y_hbm.at[pl.ds(s * (N // NS), N // NS)],
                          ytab.at[pl.ds(s * (N // NS), N // NS)],
                          semy).start()

    for r in range(16):
        for k in range(H // 16):
            zbuf[r, pl.ds(16 * k, 16)] = jnp.zeros((16,), jnp.float32)

    def zloop(j, carry):
        pltpu.sync_copy(zbuf, acc.at[pl.ds(s * RPT + 16 * j, 16)])
        return carry

    lax.fori_loop(0, RPT // 16, zloop, 0)
    pltpu.make_async_copy(y_hbm.at[pl.ds(s * (N // NS), N // NS)],
                          ytab.at[pl.ds(s * (N // NS), N // NS)],
                          semy).wait()
    plsc.subcore_barrier()

    wbase = w * CPW
    pltpu.sync_copy(ep_hbm.at[pl.ds(wbase, 2)], ib0)
    pltpu.make_async_copy(ep_hbm.at[pl.ds(wbase + 2, 2)], ib1, semi1).start()
    pltpu.make_async_copy(ytab.at[ib0.at[0].at[0]], rb0, sem0).start()

    def body(u, carry):
        a = wbase + 4 * u
        pltpu.make_async_copy(ytab.at[ib0.at[1].at[0]], rb1, sem1).start()
        pltpu.make_async_copy(ytab.at[ib0.at[0].at[0]], rb0, sem0).wait()
        pltpu.sync_copy(rb0, acc.at[ib0.at[0].at[1]], add=True)
        pltpu.make_async_copy(ep_hbm.at[pl.ds(a + 2, 2)], ib1, semi1).wait()
        pltpu.make_async_copy(ytab.at[ib1.at[0].at[0]], rb0, sem0).start()
        pltpu.make_async_copy(ytab.at[ib0.at[1].at[0]], rb1, sem1).wait()
        pltpu.sync_copy(rb1, acc.at[ib0.at[1].at[1]], add=True)
        pltpu.make_async_copy(ep_hbm.at[pl.ds(a + 4, 2)], ib0, semi0).start()
        pltpu.make_async_copy(ytab.at[ib1.at[1].at[0]], rb1, sem1).start()
        pltpu.make_async_copy(ytab.at[ib1.at[0].at[0]], rb0, sem0).wait()
        pltpu.sync_copy(rb0, acc.at[ib1.at[0].at[1]], add=True)
        pltpu.make_async_copy(ep_hbm.at[pl.ds(a + 4, 2)], ib0, semi0).wait()
        pltpu.make_async_copy(ytab.at[ib0.at[0].at[0]], rb0, sem0).start()
        pltpu.make_async_copy(ytab.at[ib1.at[1].at[0]], rb1, sem1).wait()
        pltpu.sync_copy(rb1, acc.at[ib1.at[1].at[1]], add=True)
        pltpu.make_async_copy(ep_hbm.at[pl.ds(a + 6, 2)], ib1, semi1).start()
        return carry

    lax.fori_loop(0, CPW // 4, body, 0)
    pltpu.make_async_copy(ytab.at[ib0.at[0].at[0]], rb0, sem0).wait()
    pltpu.make_async_copy(ep_hbm.at[pl.ds(wbase + CPW + 2, 2)], ib1,
                          semi1).wait()
    plsc.subcore_barrier()
    pltpu.sync_copy(acc.at[pl.ds(s * RPT, RPT)],
                    out_hbm.at[c].at[pl.ds(s * RPT, RPT)])


@functools.partial(jax.jit)
def _scatter_call(y, ep):
    return pl.kernel(
        _scatter_body,
        out_type=jax.ShapeDtypeStruct((NC, ROWS, H), jnp.float32),
        mesh=plsc.VectorSubcoreMesh(**_MESH),
        scratch_types=[
            pltpu.VMEM((2, 2, CHUNK), jnp.int32),
            pltpu.VMEM((2, 2, CHUNK), jnp.int32),
            pltpu.VMEM((CHUNK, H), jnp.float32),
            pltpu.VMEM((CHUNK, H), jnp.float32),
            pltpu.VMEM((16, H), jnp.float32),
            pltpu.VMEM_SHARED((ROWS, H), jnp.float32),
            pltpu.VMEM_SHARED((N, H), jnp.float32),
            pltpu.SemaphoreType.DMA,
            pltpu.SemaphoreType.DMA,
            pltpu.SemaphoreType.DMA,
            pltpu.SemaphoreType.DMA,
            pltpu.SemaphoreType.DMA,
        ],
        compiler_params=pltpu.CompilerParams(use_tc_tiling_on_sc=False),
    )(y, ep)


RB = 2000
NBLK = N // RB


def _t1_body(x_ref, w1_ref, degt_ref, y_ref, z_ref):
    deg = degt_ref[:, 0:1] + degt_ref[:, 1:2] + 1.0
    dis = lax.rsqrt(deg)
    xw = jnp.dot(x_ref[...], w1_ref[...], preferred_element_type=jnp.float32)
    y_ref[...] = xw * dis
    z_ref[...] = xw / deg


def _t1(x, W1, degt):
    return pl.pallas_call(
        _t1_body,
        grid=(NBLK,),
        in_specs=[
            pl.BlockSpec((RB, DIN), lambda i: (i, 0)),
            pl.BlockSpec((DIN, H), lambda i: (0, 0)),
            pl.BlockSpec((RB, 2), lambda i: (i, 0)),
        ],
        out_specs=[
            pl.BlockSpec((RB, H), lambda i: (i, 0)),
            pl.BlockSpec((RB, H), lambda i: (i, 0)),
        ],
        out_shape=[
            jax.ShapeDtypeStruct((N, H), jnp.float32),
            jax.ShapeDtypeStruct((N, H), jnp.float32),
        ],
    )(x, W1, degt)


def _t2_body(acc_ref, z1_ref, degt_ref, w2_ref, b1_ref, y_ref, z_ref):
    deg = degt_ref[:, 0:1] + degt_ref[:, 1:2] + 1.0
    dis = lax.rsqrt(deg)
    a = acc_ref[0] + acc_ref[1]
    h1 = jnp.maximum(a * dis + z1_ref[...] + b1_ref[...], 0.0)
    xw = jnp.dot(h1, w2_ref[...], preferred_element_type=jnp.float32)
    y_ref[...] = xw * dis
    z_ref[...] = xw / deg


def _t2(acc1, z1, degt, W2, b1r):
    return pl.pallas_call(
        _t2_body,
        grid=(NBLK,),
        in_specs=[
            pl.BlockSpec((NC, RB, H), lambda i: (0, i, 0)),
            pl.BlockSpec((RB, H), lambda i: (i, 0)),
            pl.BlockSpec((RB, 2), lambda i: (i, 0)),
            pl.BlockSpec((H, H), lambda i: (0, 0)),
            pl.BlockSpec((1, H), lambda i: (0, 0)),
        ],
        out_specs=[
            pl.BlockSpec((RB, H), lambda i: (i, 0)),
            pl.BlockSpec((RB, H), lambda i: (i, 0)),
        ],
        out_shape=[
            jax.ShapeDtypeStruct((N, H), jnp.float32),
            jax.ShapeDtypeStruct((N, H), jnp.float32),
        ],
    )(acc1, z1, degt, W2, b1r)


def _t3_body(acc_ref, z2_ref, degt_ref, bcol_ref, b2_ref, wl1_ref, bl1_ref,
             wl2_ref, bl2_ref, out_ref, gsum, cnt):
    i = pl.program_id(0)

    @pl.when(i == 0)
    def _():
        gsum[...] = jnp.zeros((G, H), jnp.float32)
        cnt[...] = jnp.zeros((G, 1), jnp.float32)

    deg = degt_ref[:, 0:1] + degt_ref[:, 1:2] + 1.0
    dis = lax.rsqrt(deg)
    h2 = (acc_ref[0] + acc_ref[1]) * dis + z2_ref[...] + b2_ref[...]
    pt = (bcol_ref[...] == lax.broadcasted_iota(jnp.int32, (1, G), 1))
    pt = pt.astype(jnp.float32)
    dn = (((0,), (0,)), ((), ()))
    gsum[...] += lax.dot_general(pt, h2, dn,
                                 preferred_element_type=jnp.float32)
    cnt[...] += lax.dot_general(pt, jnp.ones((RB, 1), jnp.float32), dn,
                                preferred_element_type=jnp.float32)

    @pl.when(i == NBLK - 1)
    def _():
        g = gsum[...] / jnp.maximum(cnt[...], 1.0)
        g = jnp.dot(g, wl1_ref[...],
                    preferred_element_type=jnp.float32) + bl1_ref[...]
        g = jnp.dot(g, wl2_ref[...],
                    preferred_element_type=jnp.float32) + bl2_ref[...]
        m = jnp.max(g, axis=0, keepdims=True)
        e = jnp.exp(g - m)
        out_ref[...] = e / jnp.sum(e, axis=0, keepdims=True)


def _t3(acc2, z2, degt, bcol, b2r, Wl1, bl1r, Wl2, bl2r):
    return pl.pallas_call(
        _t3_body,
        grid=(NBLK,),
        in_specs=[
            pl.BlockSpec((NC, RB, H), lambda i: (0, i, 0)),
            pl.BlockSpec((RB, H), lambda i: (i, 0)),
            pl.BlockSpec((RB, 2), lambda i: (i, 0)),
            pl.BlockSpec((RB, 1), lambda i: (i, 0)),
            pl.BlockSpec((1, H), lambda i: (0, 0)),
            pl.BlockSpec((H, 32), lambda i: (0, 0)),
            pl.BlockSpec((1, 32), lambda i: (0, 0)),
            pl.BlockSpec((32, 2), lambda i: (0, 0)),
            pl.BlockSpec((1, 2), lambda i: (0, 0)),
        ],
        out_specs=pl.BlockSpec((G, 2), lambda i: (0, 0)),
        out_shape=jax.ShapeDtypeStruct((G, 2), jnp.float32),
        scratch_shapes=[
            pltpu.VMEM((G, H), jnp.float32),
            pltpu.VMEM((G, 1), jnp.float32),
        ],
    )(acc2, z2, degt, bcol, b2r, Wl1, bl1r, Wl2, bl2r)


def kernel(x, edge_index, batch, W1, b1, W2, b2, Wl1, bl1, Wl2, bl2):
    src = edge_index[0].astype(jnp.int32)
    dst = edge_index[1].astype(jnp.int32)
    pad = EPAD + 4 * CHUNK - E
    src_p = jnp.concatenate([src, jnp.zeros((pad,), jnp.int32)])
    dst_p = jnp.concatenate([dst, jnp.full((pad,), N, jnp.int32)])
    ep = jnp.stack([src_p.reshape(NCH + 4, CHUNK),
                    dst_p.reshape(NCH + 4, CHUNK)], axis=1)

    degp = _deg_call(ep)
    degt = degp.T[:N]

    y1, z1 = _t1(x, W1, degt)
    acc1 = _scatter_call(y1, ep)
    y2, z2 = _t2(acc1, z1, degt, W2, b1.reshape(1, H))
    acc2 = _scatter_call(y2, ep)
    return _t3(acc2, z2, degt, batch.astype(jnp.int32).reshape(N, 1),
               b2.reshape(1, H), Wl1, bl1.reshape(1, 32),
               Wl2, bl2.reshape(1, 2))


# speedup vs baseline: 36.9989x; 1.0814x over previous
import functools

import jax
import jax.numpy as jnp
from jax import lax
from jax.experimental import pallas as pl
from jax.experimental.pallas import tpu as pltpu
from jax.experimental.pallas import tpu_sc as plsc

N = 10000
E = 320000
DIN = 128
H = 64
G = 128

NC = 2
NS = 16
NW = NC * NS
CHUNK = 128
CPW = 80
NCH = NW * CPW
EPAD = NCH * CHUNK
ROWS = 10240
RPT = ROWS // NS

_MESH = dict(core_axis_name="c", subcore_axis_name="s",
             num_cores=NC, num_subcores=NS)


def _zero_f32_ref(ref, n):
    for k in range(n // 16):
        ref[pl.ds(16 * k, 16)] = jnp.zeros((16,), jnp.float32)


def _deg_body(ep_hbm, out_hbm, ib0, ib1, dhist, tbuf, tmp, slab,
              semi0, semi1):
    c = lax.axis_index("c")
    s = lax.axis_index("s")
    w = c * NS + s

    def zero_hist(j, carry):
        dhist[pl.ds(16 * j, 16)] = jnp.zeros((16,), jnp.float32)
        return carry

    lax.fori_loop(0, ROWS // 16, zero_hist, 0)

    ones = jnp.ones((16,), jnp.float32)
    wbase = w * CPW
    pltpu.sync_copy(ep_hbm.at[pl.ds(wbase, 2)], ib0)
    pltpu.make_async_copy(ep_hbm.at[pl.ds(wbase + 2, 2)], ib1, semi1).start()

    def hist_pair(ib):
        for p in range(2):
            for k in range(CHUNK // 16):
                idx = ib[p, 1, pl.ds(16 * k, 16)]
                plsc.addupdate_scatter(dhist, [idx], ones)

    def body(u, carry):
        a = wbase + 4 * u
        hist_pair(ib0)
        pltpu.make_async_copy(ep_hbm.at[pl.ds(a + 2, 2)], ib1, semi1).wait()
        pltpu.make_async_copy(ep_hbm.at[pl.ds(a + 4, 2)], ib0, semi0).start()
        hist_pair(ib1)
        pltpu.make_async_copy(ep_hbm.at[pl.ds(a + 4, 2)], ib0, semi0).wait()
        pltpu.make_async_copy(ep_hbm.at[pl.ds(a + 6, 2)], ib1, semi1).start()
        return carry

    lax.fori_loop(0, CPW // 4, body, 0)
    pltpu.make_async_copy(ep_hbm.at[pl.ds(wbase + CPW + 2, 2)], ib1,
                          semi1).wait()

    pltpu.sync_copy(dhist, slab.at[s])
    plsc.subcore_barrier()
    pltpu.sync_copy(slab.at[:, pl.ds(s * RPT, RPT)], tbuf)

    def red(j, carry):
        o = 16 * j
        v = tbuf[0, pl.ds(o, 16)]
        for k in range(1, NS):
            v = v + tbuf[k, pl.ds(o, 16)]
        tmp[pl.ds(o, 16)] = v
        return carry

    lax.fori_loop(0, RPT // 16, red, 0)
    pltpu.sync_copy(tmp, out_hbm.at[c].at[pl.ds(s * RPT, RPT)])


@functools.partial(jax.jit)
def _deg_call(ep):
    return pl.kernel(
        _deg_body,
        out_type=jax.ShapeDtypeStruct((NC, ROWS), jnp.float32),
        mesh=plsc.VectorSubcoreMesh(**_MESH),
        scratch_types=[
            pltpu.VMEM((2, 2, CHUNK), jnp.int32),
            pltpu.VMEM((2, 2, CHUNK), jnp.int32),
            pltpu.VMEM((ROWS,), jnp.float32),
            pltpu.VMEM((NS, RPT), jnp.float32),
            pltpu.VMEM((RPT,), jnp.float32),
            pltpu.VMEM_SHARED((NS, ROWS), jnp.float32),
            pltpu.SemaphoreType.DMA,
            pltpu.SemaphoreType.DMA,
        ],
        compiler_params=pltpu.CompilerParams(needs_layout_passes=False),
    )(ep)


def _scatter_body(y_hbm, ep_hbm, out_hbm, ib0, ib1, rb0, rb1, rb2, rb3,
                  zbuf, acc, ytab, sg0, sg1, sg2, sg3, ss0, ss1, ss2, ss3,
                  semi0, semi1, semy):
    c = lax.axis_index("c")
    s = lax.axis_index("s")
    w = c * NS + s

    pltpu.make_async_copy(y_hbm.at[pl.ds(s * (N // NS), N // NS)],
                          ytab.at[pl.ds(s * (N // NS), N // NS)],
                          semy).start()

    for r in range(16):
        for k in range(H // 16):
            zbuf[r, pl.ds(16 * k, 16)] = jnp.zeros((16,), jnp.float32)

    def zloop(j, carry):
        pltpu.sync_copy(zbuf, acc.at[pl.ds(s * RPT + 16 * j, 16)])
        return carry

    lax.fori_loop(0, RPT // 16, zloop, 0)
    pltpu.make_async_copy(y_hbm.at[pl.ds(s * (N // NS), N // NS)],
                          ytab.at[pl.ds(s * (N // NS), N // NS)],
                          semy).wait()
    plsc.subcore_barrier()

    wbase = w * CPW
    rb = (rb0, rb1, rb2, rb3)
    sg = (sg0, sg1, sg2, sg3)
    ss = (ss0, ss1, ss2, ss3)

    def gath(ib, k, r):
        return pltpu.make_async_copy(ytab.at[ib.at[k].at[0]], rb[r], sg[r])

    def quarter(ibX, ibY, c, ldX, ldY):
        gath(ibX, 0, 0).wait()
        d0 = pltpu.async_copy(rb0, acc.at[ibX.at[0].at[1]], ss0, add=True)
        gath(ibX, 2, 2).start()
        gath(ibX, 1, 1).wait()
        d1 = pltpu.async_copy(rb1, acc.at[ibX.at[1].at[1]], ss1, add=True)
        gath(ibX, 3, 3).start()
        pltpu.make_async_copy(ep_hbm.at[pl.ds(c + 4, 4)], ibY, ldY).wait()
        d0.wait()
        gath(ibY, 0, 0).start()
        gath(ibX, 2, 2).wait()
        d2 = pltpu.async_copy(rb2, acc.at[ibX.at[2].at[1]], ss2, add=True)
        d1.wait()
        gath(ibY, 1, 1).start()
        gath(ibX, 3, 3).wait()
        d3 = pltpu.async_copy(rb3, acc.at[ibX.at[3].at[1]], ss3, add=True)
        d2.wait()
        d3.wait()
        pltpu.make_async_copy(ep_hbm.at[pl.ds(c + 8, 4)], ibX, ldX).start()

    pltpu.sync_copy(ep_hbm.at[pl.ds(wbase, 4)], ib0)
    pltpu.make_async_copy(ep_hbm.at[pl.ds(wbase + 4, 4)], ib1, semi1).start()
    gath(ib0, 0, 0).start()
    gath(ib0, 1, 1).start()

    def body(u, carry):
        c = wbase + 8 * u
        quarter(ib0, ib1, c, semi0, semi1)
        quarter(ib1, ib0, c + 4, semi1, semi0)
        return carry

    lax.fori_loop(0, CPW // 8, body, 0)
    gath(ib0, 0, 0).wait()
    gath(ib0, 1, 1).wait()
    pltpu.make_async_copy(ep_hbm.at[pl.ds(wbase + CPW + 4, 4)], ib1,
                          semi1).wait()
    plsc.subcore_barrier()
    pltpu.sync_copy(acc.at[pl.ds(s * RPT, RPT)],
                    out_hbm.at[c].at[pl.ds(s * RPT, RPT)])


@functools.partial(jax.jit)
def _scatter_call(y, ep):
    return pl.kernel(
        _scatter_body,
        out_type=jax.ShapeDtypeStruct((NC, ROWS, H), jnp.float32),
        mesh=plsc.VectorSubcoreMesh(**_MESH),
        scratch_types=(
            [pltpu.VMEM((4, 2, CHUNK), jnp.int32)] * 2
            + [pltpu.VMEM((CHUNK, H), jnp.float32)] * 4
            + [pltpu.VMEM((16, H), jnp.float32),
               pltpu.VMEM_SHARED((ROWS, H), jnp.float32),
               pltpu.VMEM_SHARED((N, H), jnp.float32)]
            + [pltpu.SemaphoreType.DMA] * 11
        ),
        compiler_params=pltpu.CompilerParams(use_tc_tiling_on_sc=False),
    )(y, ep)


RB = 2000
NBLK = N // RB


def _t1_body(x_ref, w1_ref, degt_ref, y_ref, z_ref):
    deg = degt_ref[:, 0:1] + degt_ref[:, 1:2] + 1.0
    dis = lax.rsqrt(deg)
    xw = jnp.dot(x_ref[...], w1_ref[...], preferred_element_type=jnp.float32)
    y_ref[...] = xw * dis
    z_ref[...] = xw / deg


def _t1(x, W1, degt):
    return pl.pallas_call(
        _t1_body,
        grid=(NBLK,),
        in_specs=[
            pl.BlockSpec((RB, DIN), lambda i: (i, 0)),
            pl.BlockSpec((DIN, H), lambda i: (0, 0)),
            pl.BlockSpec((RB, 2), lambda i: (i, 0)),
        ],
        out_specs=[
            pl.BlockSpec((RB, H), lambda i: (i, 0)),
            pl.BlockSpec((RB, H), lambda i: (i, 0)),
        ],
        out_shape=[
            jax.ShapeDtypeStruct((N, H), jnp.float32),
            jax.ShapeDtypeStruct((N, H), jnp.float32),
        ],
    )(x, W1, degt)


def _t2_body(acc_ref, z1_ref, degt_ref, w2_ref, b1_ref, y_ref, z_ref):
    deg = degt_ref[:, 0:1] + degt_ref[:, 1:2] + 1.0
    dis = lax.rsqrt(deg)
    a = acc_ref[0] + acc_ref[1]
    h1 = jnp.maximum(a * dis + z1_ref[...] + b1_ref[...], 0.0)
    xw = jnp.dot(h1, w2_ref[...], preferred_element_type=jnp.float32)
    y_ref[...] = xw * dis
    z_ref[...] = xw / deg


def _t2(acc1, z1, degt, W2, b1r):
    return pl.pallas_call(
        _t2_body,
        grid=(NBLK,),
        in_specs=[
            pl.BlockSpec((NC, RB, H), lambda i: (0, i, 0)),
            pl.BlockSpec((RB, H), lambda i: (i, 0)),
            pl.BlockSpec((RB, 2), lambda i: (i, 0)),
            pl.BlockSpec((H, H), lambda i: (0, 0)),
            pl.BlockSpec((1, H), lambda i: (0, 0)),
        ],
        out_specs=[
            pl.BlockSpec((RB, H), lambda i: (i, 0)),
            pl.BlockSpec((RB, H), lambda i: (i, 0)),
        ],
        out_shape=[
            jax.ShapeDtypeStruct((N, H), jnp.float32),
            jax.ShapeDtypeStruct((N, H), jnp.float32),
        ],
    )(acc1, z1, degt, W2, b1r)


def _t3_body(acc_ref, z2_ref, degt_ref, bcol_ref, b2_ref, wl1_ref, bl1_ref,
             wl2_ref, bl2_ref, out_ref, gsum, cnt):
    i = pl.program_id(0)

    @pl.when(i == 0)
    def _():
        gsum[...] = jnp.zeros((G, H), jnp.float32)
        cnt[...] = jnp.zeros((G, 1), jnp.float32)

    deg = degt_ref[:, 0:1] + degt_ref[:, 1:2] + 1.0
    dis = lax.rsqrt(deg)
    h2 = (acc_ref[0] + acc_ref[1]) * dis + z2_ref[...] + b2_ref[...]
    pt = (bcol_ref[...] == lax.broadcasted_iota(jnp.int32, (1, G), 1))
    pt = pt.astype(jnp.float32)
    dn = (((0,), (0,)), ((), ()))
    gsum[...] += lax.dot_general(pt, h2, dn,
                                 preferred_element_type=jnp.float32)
    cnt[...] += lax.dot_general(pt, jnp.ones((RB, 1), jnp.float32), dn,
                                preferred_element_type=jnp.float32)

    @pl.when(i == NBLK - 1)
    def _():
        g = gsum[...] / jnp.maximum(cnt[...], 1.0)
        g = jnp.dot(g, wl1_ref[...],
                    preferred_element_type=jnp.float32) + bl1_ref[...]
        g = jnp.dot(g, wl2_ref[...],
                    preferred_element_type=jnp.float32) + bl2_ref[...]
        m = jnp.max(g, axis=0, keepdims=True)
        e = jnp.exp(g - m)
        out_ref[...] = e / jnp.sum(e, axis=0, keepdims=True)


def _t3(acc2, z2, degt, bcol, b2r, Wl1, bl1r, Wl2, bl2r):
    return pl.pallas_call(
        _t3_body,
        grid=(NBLK,),
        in_specs=[
            pl.BlockSpec((NC, RB, H), lambda i: (0, i, 0)),
            pl.BlockSpec((RB, H), lambda i: (i, 0)),
            pl.BlockSpec((RB, 2), lambda i: (i, 0)),
            pl.BlockSpec((RB, 1), lambda i: (i, 0)),
            pl.BlockSpec((1, H), lambda i: (0, 0)),
            pl.BlockSpec((H, 32), lambda i: (0, 0)),
            pl.BlockSpec((1, 32), lambda i: (0, 0)),
            pl.BlockSpec((32, 2), lambda i: (0, 0)),
            pl.BlockSpec((1, 2), lambda i: (0, 0)),
        ],
        out_specs=pl.BlockSpec((G, 2), lambda i: (0, 0)),
        out_shape=jax.ShapeDtypeStruct((G, 2), jnp.float32),
        scratch_shapes=[
            pltpu.VMEM((G, H), jnp.float32),
            pltpu.VMEM((G, 1), jnp.float32),
        ],
    )(acc2, z2, degt, bcol, b2r, Wl1, bl1r, Wl2, bl2r)


def kernel(x, edge_index, batch, W1, b1, W2, b2, Wl1, bl1, Wl2, bl2):
    src = edge_index[0].astype(jnp.int32)
    dst = edge_index[1].astype(jnp.int32)
    pad = EPAD + 8 * CHUNK - E
    src_p = jnp.concatenate([src, jnp.zeros((pad,), jnp.int32)])
    dst_p = jnp.concatenate([dst, jnp.full((pad,), N, jnp.int32)])
    ep = jnp.stack([src_p.reshape(NCH + 8, CHUNK),
                    dst_p.reshape(NCH + 8, CHUNK)], axis=1)

    degp = _deg_call(ep)
    degt = degp.T[:N]

    y1, z1 = _t1(x, W1, degt)
    acc1 = _scatter_call(y1, ep)
    y2, z2 = _t2(acc1, z1, degt, W2, b1.reshape(1, H))
    acc2 = _scatter_call(y2, ep)
    return _t3(acc2, z2, degt, batch.astype(jnp.int32).reshape(N, 1),
               b2.reshape(1, H), Wl1, bl1.reshape(1, 32),
               Wl2, bl2.reshape(1, 2))


# speedup vs baseline: 40.3174x; 1.0897x over previous
import functools

import jax
import jax.numpy as jnp
from jax import lax
from jax.experimental import pallas as pl
from jax.experimental.pallas import tpu as pltpu
from jax.experimental.pallas import tpu_sc as plsc

N = 10000
E = 320000
DIN = 128
H = 64
G = 128

NC = 2
NS = 16
NW = NC * NS
CHUNK = 128
CPW = 80
NCH = NW * CPW
EPAD = NCH * CHUNK
ROWS = 10240
RPT = ROWS // NS

_MESH = dict(core_axis_name="c", subcore_axis_name="s",
             num_cores=NC, num_subcores=NS)


def _zero_f32_ref(ref, n):
    for k in range(n // 16):
        ref[pl.ds(16 * k, 16)] = jnp.zeros((16,), jnp.float32)


def _deg_body(ei_hbm, out_hbm, dbuf, dhist, tbuf, tmp, slab, semi):
    c = lax.axis_index("c")
    s = lax.axis_index("s")
    w = c * NS + s
    epw = (E // (NW * CHUNK)) * CHUNK

    pltpu.make_async_copy(ei_hbm.at[1].at[pl.ds(w * epw, epw)], dbuf,
                          semi).start()

    def zero_hist(j, carry):
        dhist[pl.ds(16 * j, 16)] = jnp.zeros((16,), jnp.float32)
        return carry

    lax.fori_loop(0, ROWS // 16, zero_hist, 0)
    pltpu.make_async_copy(ei_hbm.at[1].at[pl.ds(w * epw, epw)], dbuf,
                          semi).wait()

    ones = jnp.ones((16,), jnp.float32)

    def hist(j, carry):
        idx = dbuf[pl.ds(16 * j, 16)]
        plsc.addupdate_scatter(dhist, [idx], ones)
        return carry

    lax.fori_loop(0, epw // 16, hist, 0)

    @pl.when(w < 4)
    def _():
        pltpu.sync_copy(
            ei_hbm.at[1].at[pl.ds(NW * epw + w * CHUNK, CHUNK)],
            dbuf.at[pl.ds(0, CHUNK)])
        lax.fori_loop(0, CHUNK // 16, hist, 0)

    pltpu.sync_copy(dhist, slab.at[s])
    plsc.subcore_barrier()
    pltpu.sync_copy(slab.at[:, pl.ds(s * RPT, RPT)], tbuf)

    def red(j, carry):
        o = 16 * j
        v = tbuf[0, pl.ds(o, 16)]
        for k in range(1, NS):
            v = v + tbuf[k, pl.ds(o, 16)]
        tmp[pl.ds(o, 16)] = v
        return carry

    lax.fori_loop(0, RPT // 16, red, 0)
    pltpu.sync_copy(tmp, out_hbm.at[c].at[pl.ds(s * RPT, RPT)])


@functools.partial(jax.jit)
def _deg_call(ei):
    return pl.kernel(
        _deg_body,
        out_type=jax.ShapeDtypeStruct((NC, ROWS), jnp.float32),
        mesh=plsc.VectorSubcoreMesh(**_MESH),
        scratch_types=[
            pltpu.VMEM(((E // (NW * CHUNK)) * CHUNK,), jnp.int32),
            pltpu.VMEM((ROWS,), jnp.float32),
            pltpu.VMEM((NS, RPT), jnp.float32),
            pltpu.VMEM((RPT,), jnp.float32),
            pltpu.VMEM_SHARED((NS, ROWS), jnp.float32),
            pltpu.SemaphoreType.DMA,
        ],
        compiler_params=pltpu.CompilerParams(needs_layout_passes=False),
    )(ei)


def _scatter_body(y_hbm, ep_hbm, out_hbm, ib0, ib1, rb0, rb1, rb2, rb3,
                  zbuf, acc, ytab, sg0, sg1, sg2, sg3, ss0, ss1, ss2, ss3,
                  semi0, semi1, semy):
    c = lax.axis_index("c")
    s = lax.axis_index("s")
    w = c * NS + s

    pltpu.make_async_copy(y_hbm.at[pl.ds(s * (N // NS), N // NS)],
                          ytab.at[pl.ds(s * (N // NS), N // NS)],
                          semy).start()

    for r in range(16):
        for k in range(H // 16):
            zbuf[r, pl.ds(16 * k, 16)] = jnp.zeros((16,), jnp.float32)

    def zloop(j, carry):
        pltpu.sync_copy(zbuf, acc.at[pl.ds(s * RPT + 16 * j, 16)])
        return carry

    lax.fori_loop(0, RPT // 16, zloop, 0)
    pltpu.make_async_copy(y_hbm.at[pl.ds(s * (N // NS), N // NS)],
                          ytab.at[pl.ds(s * (N // NS), N // NS)],
                          semy).wait()
    plsc.subcore_barrier()

    wbase = w * CPW
    rb = (rb0, rb1, rb2, rb3)
    sg = (sg0, sg1, sg2, sg3)
    ss = (ss0, ss1, ss2, ss3)

    def gath(ib, k, r):
        return pltpu.make_async_copy(ytab.at[ib.at[k].at[0]], rb[r], sg[r])

    def quarter(ibX, ibY, c, ldX, ldY):
        gath(ibX, 0, 0).wait()
        d0 = pltpu.async_copy(rb0, acc.at[ibX.at[0].at[1]], ss0, add=True)
        gath(ibX, 2, 2).start()
        gath(ibX, 1, 1).wait()
        d1 = pltpu.async_copy(rb1, acc.at[ibX.at[1].at[1]], ss1, add=True)
        gath(ibX, 3, 3).start()
        pltpu.make_async_copy(ep_hbm.at[pl.ds(c + 4, 4)], ibY, ldY).wait()
        d0.wait()
        gath(ibY, 0, 0).start()
        gath(ibX, 2, 2).wait()
        d2 = pltpu.async_copy(rb2, acc.at[ibX.at[2].at[1]], ss2, add=True)
        d1.wait()
        gath(ibY, 1, 1).start()
        gath(ibX, 3, 3).wait()
        d3 = pltpu.async_copy(rb3, acc.at[ibX.at[3].at[1]], ss3, add=True)
        d2.wait()
        d3.wait()
        pltpu.make_async_copy(ep_hbm.at[pl.ds(c + 8, 4)], ibX, ldX).start()

    pltpu.sync_copy(ep_hbm.at[pl.ds(wbase, 4)], ib0)
    pltpu.make_async_copy(ep_hbm.at[pl.ds(wbase + 4, 4)], ib1, semi1).start()
    gath(ib0, 0, 0).start()
    gath(ib0, 1, 1).start()

    def body(u, carry):
        c = wbase + 8 * u
        quarter(ib0, ib1, c, semi0, semi1)
        quarter(ib1, ib0, c + 4, semi1, semi0)
        return carry

    lax.fori_loop(0, CPW // 8, body, 0)
    gath(ib0, 0, 0).wait()
    gath(ib0, 1, 1).wait()
    pltpu.make_async_copy(ep_hbm.at[pl.ds(wbase + CPW + 4, 4)], ib1,
                          semi1).wait()
    plsc.subcore_barrier()
    pltpu.sync_copy(acc.at[pl.ds(s * RPT, RPT)],
                    out_hbm.at[c].at[pl.ds(s * RPT, RPT)])


@functools.partial(jax.jit)
def _scatter_call(y, ep):
    return pl.kernel(
        _scatter_body,
        out_type=jax.ShapeDtypeStruct((NC, ROWS, H), jnp.float32),
        mesh=plsc.VectorSubcoreMesh(**_MESH),
        scratch_types=(
            [pltpu.VMEM((4, 2, CHUNK), jnp.int32)] * 2
            + [pltpu.VMEM((CHUNK, H), jnp.float32)] * 4
            + [pltpu.VMEM((16, H), jnp.float32),
               pltpu.VMEM_SHARED((ROWS, H), jnp.float32),
               pltpu.VMEM_SHARED((N, H), jnp.float32)]
            + [pltpu.SemaphoreType.DMA] * 11
        ),
        compiler_params=pltpu.CompilerParams(use_tc_tiling_on_sc=False),
    )(y, ep)


RB = 2000
NBLK = N // RB


def _t1a_body(x_ref, w1_ref, xw_ref):
    xw_ref[...] = jnp.dot(x_ref[...], w1_ref[...],
                          preferred_element_type=jnp.float32)


def _t1a(x, W1):
    return pl.pallas_call(
        _t1a_body,
        grid=(NBLK,),
        in_specs=[
            pl.BlockSpec((RB, DIN), lambda i: (i, 0)),
            pl.BlockSpec((DIN, H), lambda i: (0, 0)),
        ],
        out_specs=pl.BlockSpec((RB, H), lambda i: (i, 0)),
        out_shape=jax.ShapeDtypeStruct((N, H), jnp.float32),
    )(x, W1)


def _t1b_body(xw_ref, degt_ref, y_ref, z_ref):
    deg = degt_ref[:, 0:1] + degt_ref[:, 1:2] + 1.0
    dis = lax.rsqrt(deg)
    xw = xw_ref[...]
    y_ref[...] = xw * dis
    z_ref[...] = xw / deg


def _t1b(xw, degt):
    return pl.pallas_call(
        _t1b_body,
        grid=(NBLK,),
        in_specs=[
            pl.BlockSpec((RB, H), lambda i: (i, 0)),
            pl.BlockSpec((RB, 2), lambda i: (i, 0)),
        ],
        out_specs=[
            pl.BlockSpec((RB, H), lambda i: (i, 0)),
            pl.BlockSpec((RB, H), lambda i: (i, 0)),
        ],
        out_shape=[
            jax.ShapeDtypeStruct((N, H), jnp.float32),
            jax.ShapeDtypeStruct((N, H), jnp.float32),
        ],
    )(xw, degt)


def _t2_body(acc_ref, z1_ref, degt_ref, w2_ref, b1_ref, y_ref, z_ref):
    deg = degt_ref[:, 0:1] + degt_ref[:, 1:2] + 1.0
    dis = lax.rsqrt(deg)
    a = acc_ref[0] + acc_ref[1]
    h1 = jnp.maximum(a * dis + z1_ref[...] + b1_ref[...], 0.0)
    xw = jnp.dot(h1, w2_ref[...], preferred_element_type=jnp.float32)
    y_ref[...] = xw * dis
    z_ref[...] = xw / deg


def _t2(acc1, z1, degt, W2, b1r):
    return pl.pallas_call(
        _t2_body,
        grid=(NBLK,),
        in_specs=[
            pl.BlockSpec((NC, RB, H), lambda i: (0, i, 0)),
            pl.BlockSpec((RB, H), lambda i: (i, 0)),
            pl.BlockSpec((RB, 2), lambda i: (i, 0)),
            pl.BlockSpec((H, H), lambda i: (0, 0)),
            pl.BlockSpec((1, H), lambda i: (0, 0)),
        ],
        out_specs=[
            pl.BlockSpec((RB, H), lambda i: (i, 0)),
            pl.BlockSpec((RB, H), lambda i: (i, 0)),
        ],
        out_shape=[
            jax.ShapeDtypeStruct((N, H), jnp.float32),
            jax.ShapeDtypeStruct((N, H), jnp.float32),
        ],
    )(acc1, z1, degt, W2, b1r)


def _t3_body(acc_ref, z2_ref, degt_ref, bcol_ref, b2_ref, wl1_ref, bl1_ref,
             wl2_ref, bl2_ref, out_ref, gsum, cnt):
    i = pl.program_id(0)

    @pl.when(i == 0)
    def _():
        gsum[...] = jnp.zeros((G, H), jnp.float32)
        cnt[...] = jnp.zeros((G, 1), jnp.float32)

    deg = degt_ref[:, 0:1] + degt_ref[:, 1:2] + 1.0
    dis = lax.rsqrt(deg)
    h2 = (acc_ref[0] + acc_ref[1]) * dis + z2_ref[...] + b2_ref[...]
    pt = (bcol_ref[...] == lax.broadcasted_iota(jnp.int32, (1, G), 1))
    pt = pt.astype(jnp.float32)
    dn = (((0,), (0,)), ((), ()))
    gsum[...] += lax.dot_general(pt, h2, dn,
                                 preferred_element_type=jnp.float32)
    cnt[...] += lax.dot_general(pt, jnp.ones((RB, 1), jnp.float32), dn,
                                preferred_element_type=jnp.float32)

    @pl.when(i == NBLK - 1)
    def _():
        g = gsum[...] / jnp.maximum(cnt[...], 1.0)
        g = jnp.dot(g, wl1_ref[...],
                    preferred_element_type=jnp.float32) + bl1_ref[...]
        g = jnp.dot(g, wl2_ref[...],
                    preferred_element_type=jnp.float32) + bl2_ref[...]
        m = jnp.max(g, axis=0, keepdims=True)
        e = jnp.exp(g - m)
        out_ref[...] = e / jnp.sum(e, axis=0, keepdims=True)


def _t3(acc2, z2, degt, bcol, b2r, Wl1, bl1r, Wl2, bl2r):
    return pl.pallas_call(
        _t3_body,
        grid=(NBLK,),
        in_specs=[
            pl.BlockSpec((NC, RB, H), lambda i: (0, i, 0)),
            pl.BlockSpec((RB, H), lambda i: (i, 0)),
            pl.BlockSpec((RB, 2), lambda i: (i, 0)),
            pl.BlockSpec((RB, 1), lambda i: (i, 0)),
            pl.BlockSpec((1, H), lambda i: (0, 0)),
            pl.BlockSpec((H, 32), lambda i: (0, 0)),
            pl.BlockSpec((1, 32), lambda i: (0, 0)),
            pl.BlockSpec((32, 2), lambda i: (0, 0)),
            pl.BlockSpec((1, 2), lambda i: (0, 0)),
        ],
        out_specs=pl.BlockSpec((G, 2), lambda i: (0, 0)),
        out_shape=jax.ShapeDtypeStruct((G, 2), jnp.float32),
        scratch_shapes=[
            pltpu.VMEM((G, H), jnp.float32),
            pltpu.VMEM((G, 1), jnp.float32),
        ],
    )(acc2, z2, degt, bcol, b2r, Wl1, bl1r, Wl2, bl2r)


def kernel(x, edge_index, batch, W1, b1, W2, b2, Wl1, bl1, Wl2, bl2):
    ei32 = edge_index.astype(jnp.int32)
    src = ei32[0]
    dst = ei32[1]
    pad = EPAD + 8 * CHUNK - E
    src_p = jnp.concatenate([src, jnp.zeros((pad,), jnp.int32)])
    dst_p = jnp.concatenate([dst, jnp.full((pad,), N, jnp.int32)])
    ep = jnp.stack([src_p.reshape(NCH + 8, CHUNK),
                    dst_p.reshape(NCH + 8, CHUNK)], axis=1)

    degp = _deg_call(ei32)
    degt = degp.T[:N]

    xw1 = _t1a(x, W1)
    y1, z1 = _t1b(xw1, degt)
    acc1 = _scatter_call(y1, ep)
    y2, z2 = _t2(acc1, z1, degt, W2, b1.reshape(1, H))
    acc2 = _scatter_call(y2, ep)
    return _t3(acc2, z2, degt, batch.astype(jnp.int32).reshape(N, 1),
               b2.reshape(1, H), Wl1, bl1.reshape(1, 32),
               Wl2, bl2.reshape(1, 2))


# speedup vs baseline: 40.6163x; 1.0074x over previous
import functools

import jax
import jax.numpy as jnp
from jax import lax
from jax.experimental import pallas as pl
from jax.experimental.pallas import tpu as pltpu
from jax.experimental.pallas import tpu_sc as plsc

N = 10000
E = 320000
DIN = 128
H = 64
G = 128

NC = 2
NS = 16
NW = NC * NS
CHUNK = 128
CPW = 80
NCH = NW * CPW
EPAD = NCH * CHUNK
ROWS = 10240
RPT = ROWS // NS

_MESH = dict(core_axis_name="c", subcore_axis_name="s",
             num_cores=NC, num_subcores=NS)


def _zero_f32_ref(ref, n):
    for k in range(n // 16):
        ref[pl.ds(16 * k, 16)] = jnp.zeros((16,), jnp.float32)


def _deg_body(ei_hbm, out_hbm, dbuf, dhist, tbuf, tmp, slab, semi):
    c = lax.axis_index("c")
    s = lax.axis_index("s")
    w = c * NS + s
    epw = (E // (NW * CHUNK)) * CHUNK

    pltpu.make_async_copy(ei_hbm.at[1].at[pl.ds(w * epw, epw)], dbuf,
                          semi).start()

    def zero_hist(j, carry):
        dhist[pl.ds(16 * j, 16)] = jnp.zeros((16,), jnp.float32)
        return carry

    lax.fori_loop(0, ROWS // 16, zero_hist, 0)
    pltpu.make_async_copy(ei_hbm.at[1].at[pl.ds(w * epw, epw)], dbuf,
                          semi).wait()

    ones = jnp.ones((16,), jnp.float32)

    def hist(j, carry):
        idx = dbuf[pl.ds(16 * j, 16)]
        plsc.addupdate_scatter(dhist, [idx], ones)
        return carry

    lax.fori_loop(0, epw // 16, hist, 0)

    @pl.when(w < 4)
    def _():
        pltpu.sync_copy(
            ei_hbm.at[1].at[pl.ds(NW * epw + w * CHUNK, CHUNK)],
            dbuf.at[pl.ds(0, CHUNK)])
        lax.fori_loop(0, CHUNK // 16, hist, 0)

    pltpu.sync_copy(dhist, slab.at[s])
    plsc.subcore_barrier()
    pltpu.sync_copy(slab.at[:, pl.ds(s * RPT, RPT)], tbuf)

    def red(j, carry):
        o = 16 * j
        v = tbuf[0, pl.ds(o, 16)]
        for k in range(1, NS):
            v = v + tbuf[k, pl.ds(o, 16)]
        tmp[pl.ds(o, 16)] = v
        return carry

    lax.fori_loop(0, RPT // 16, red, 0)
    pltpu.sync_copy(tmp, out_hbm.at[c].at[pl.ds(s * RPT, RPT)])


@functools.partial(jax.jit)
def _deg_call(ei):
    return pl.kernel(
        _deg_body,
        out_type=jax.ShapeDtypeStruct((NC, ROWS), jnp.float32),
        mesh=plsc.VectorSubcoreMesh(**_MESH),
        scratch_types=[
            pltpu.VMEM(((E // (NW * CHUNK)) * CHUNK,), jnp.int32),
            pltpu.VMEM((ROWS,), jnp.float32),
            pltpu.VMEM((NS, RPT), jnp.float32),
            pltpu.VMEM((RPT,), jnp.float32),
            pltpu.VMEM_SHARED((NS, ROWS), jnp.float32),
            pltpu.SemaphoreType.DMA,
        ],
        compiler_params=pltpu.CompilerParams(needs_layout_passes=False),
    )(ei)


def _scatter_body(y_hbm, sr_hbm, dr_hbm, out_hbm, ib0s, ib0d, ib1s, ib1d,
                  rb0, rb1, rb2, rb3, zbuf, acc, ytab,
                  sg0, sg1, sg2, sg3, ss0, ss1, ss2, ss3,
                  si0s, si0d, si1s, si1d, semy):
    c = lax.axis_index("c")
    s = lax.axis_index("s")
    w = c * NS + s

    pltpu.make_async_copy(y_hbm.at[pl.ds(s * (N // NS), N // NS)],
                          ytab.at[pl.ds(s * (N // NS), N // NS)],
                          semy).start()

    for r in range(16):
        for k in range(H // 16):
            zbuf[r, pl.ds(16 * k, 16)] = jnp.zeros((16,), jnp.float32)

    def zloop(j, carry):
        pltpu.sync_copy(zbuf, acc.at[pl.ds(s * RPT + 16 * j, 16)])
        return carry

    lax.fori_loop(0, RPT // 16, zloop, 0)
    pltpu.make_async_copy(y_hbm.at[pl.ds(s * (N // NS), N // NS)],
                          ytab.at[pl.ds(s * (N // NS), N // NS)],
                          semy).wait()
    plsc.subcore_barrier()

    wbase = w * CPW
    rb = (rb0, rb1, rb2, rb3)
    sg = (sg0, sg1, sg2, sg3)

    def gath(ibs, k, r):
        return pltpu.make_async_copy(ytab.at[ibs.at[k]], rb[r], sg[r])

    def lds(ibs, ibd, c, lds_, ldd_):
        return (pltpu.make_async_copy(sr_hbm.at[pl.ds(c, 4)], ibs, lds_),
                pltpu.make_async_copy(dr_hbm.at[pl.ds(c, 4)], ibd, ldd_))

    def quarter(ibXs, ibXd, ibYs, ibYd, c, ldX, ldY):
        gath(ibXs, 0, 0).wait()
        d0 = pltpu.async_copy(rb0, acc.at[ibXd.at[0]], ss0, add=True)
        gath(ibXs, 2, 2).start()
        gath(ibXs, 1, 1).wait()
        d1 = pltpu.async_copy(rb1, acc.at[ibXd.at[1]], ss1, add=True)
        gath(ibXs, 3, 3).start()
        for cp in lds(ibYs, ibYd, c + 4, *ldY):
            cp.wait()
        d0.wait()
        gath(ibYs, 0, 0).start()
        gath(ibXs, 2, 2).wait()
        d2 = pltpu.async_copy(rb2, acc.at[ibXd.at[2]], ss2, add=True)
        d1.wait()
        gath(ibYs, 1, 1).start()
        gath(ibXs, 3, 3).wait()
        d3 = pltpu.async_copy(rb3, acc.at[ibXd.at[3]], ss3, add=True)
        d2.wait()
        d3.wait()
        for cp in lds(ibXs, ibXd, c + 8, *ldX):
            cp.start()

    pltpu.sync_copy(sr_hbm.at[pl.ds(wbase, 4)], ib0s)
    pltpu.sync_copy(dr_hbm.at[pl.ds(wbase, 4)], ib0d)
    for cp in lds(ib1s, ib1d, wbase + 4, si1s, si1d):
        cp.start()
    gath(ib0s, 0, 0).start()
    gath(ib0s, 1, 1).start()

    def body(u, carry):
        c = wbase + 8 * u
        quarter(ib0s, ib0d, ib1s, ib1d, c, (si0s, si0d), (si1s, si1d))
        quarter(ib1s, ib1d, ib0s, ib0d, c + 4, (si1s, si1d), (si0s, si0d))
        return carry

    lax.fori_loop(0, CPW // 8, body, 0)
    gath(ib0s, 0, 0).wait()
    gath(ib0s, 1, 1).wait()
    for cp in lds(ib1s, ib1d, wbase + CPW + 4, si1s, si1d):
        cp.wait()
    plsc.subcore_barrier()
    pltpu.sync_copy(acc.at[pl.ds(s * RPT, RPT)],
                    out_hbm.at[c].at[pl.ds(s * RPT, RPT)])


@functools.partial(jax.jit)
def _scatter_call(y, srr, dsr):
    return pl.kernel(
        _scatter_body,
        out_type=jax.ShapeDtypeStruct((NC, ROWS, H), jnp.float32),
        mesh=plsc.VectorSubcoreMesh(**_MESH),
        scratch_types=(
            [pltpu.VMEM((4, CHUNK), jnp.int32)] * 4
            + [pltpu.VMEM((CHUNK, H), jnp.float32)] * 4
            + [pltpu.VMEM((16, H), jnp.float32),
               pltpu.VMEM_SHARED((ROWS, H), jnp.float32),
               pltpu.VMEM_SHARED((N, H), jnp.float32)]
            + [pltpu.SemaphoreType.DMA] * 13
        ),
        compiler_params=pltpu.CompilerParams(use_tc_tiling_on_sc=False),
    )(y, srr, dsr)


RB = 2000
NBLK = N // RB


def _t1a_body(x_ref, w1_ref, xw_ref):
    xw_ref[...] = jnp.dot(x_ref[...], w1_ref[...],
                          preferred_element_type=jnp.float32)


def _t1a(x, W1):
    return pl.pallas_call(
        _t1a_body,
        grid=(NBLK,),
        in_specs=[
            pl.BlockSpec((RB, DIN), lambda i: (i, 0)),
            pl.BlockSpec((DIN, H), lambda i: (0, 0)),
        ],
        out_specs=pl.BlockSpec((RB, H), lambda i: (i, 0)),
        out_shape=jax.ShapeDtypeStruct((N, H), jnp.float32),
    )(x, W1)


def _t1b_body(xw_ref, degt_ref, y_ref):
    deg = degt_ref[:, 0:1] + degt_ref[:, 1:2] + 1.0
    y_ref[...] = xw_ref[...] * lax.rsqrt(deg)


def _t1b(xw, degt):
    return pl.pallas_call(
        _t1b_body,
        grid=(NBLK,),
        in_specs=[
            pl.BlockSpec((RB, H), lambda i: (i, 0)),
            pl.BlockSpec((RB, 2), lambda i: (i, 0)),
        ],
        out_specs=pl.BlockSpec((RB, H), lambda i: (i, 0)),
        out_shape=jax.ShapeDtypeStruct((N, H), jnp.float32),
    )(xw, degt)


def _t2_body(acc_ref, xw1_ref, degt_ref, w2_ref, b1_ref, y_ref, z_ref):
    deg = degt_ref[:, 0:1] + degt_ref[:, 1:2] + 1.0
    dis = lax.rsqrt(deg)
    a = acc_ref[0] + acc_ref[1]
    h1 = jnp.maximum(a * dis + xw1_ref[...] / deg + b1_ref[...], 0.0)
    xw = jnp.dot(h1, w2_ref[...], preferred_element_type=jnp.float32)
    y_ref[...] = xw * dis
    z_ref[...] = xw / deg


def _t2(acc1, xw1, degt, W2, b1r):
    return pl.pallas_call(
        _t2_body,
        grid=(NBLK,),
        in_specs=[
            pl.BlockSpec((NC, RB, H), lambda i: (0, i, 0)),
            pl.BlockSpec((RB, H), lambda i: (i, 0)),
            pl.BlockSpec((RB, 2), lambda i: (i, 0)),
            pl.BlockSpec((H, H), lambda i: (0, 0)),
            pl.BlockSpec((1, H), lambda i: (0, 0)),
        ],
        out_specs=[
            pl.BlockSpec((RB, H), lambda i: (i, 0)),
            pl.BlockSpec((RB, H), lambda i: (i, 0)),
        ],
        out_shape=[
            jax.ShapeDtypeStruct((N, H), jnp.float32),
            jax.ShapeDtypeStruct((N, H), jnp.float32),
        ],
    )(acc1, xw1, degt, W2, b1r)


def _t3_body(acc_ref, z2_ref, degt_ref, bcol_ref, b2_ref, wl1_ref, bl1_ref,
             wl2_ref, bl2_ref, out_ref, gsum, cnt):
    i = pl.program_id(0)

    @pl.when(i == 0)
    def _():
        gsum[...] = jnp.zeros((G, H), jnp.float32)
        cnt[...] = jnp.zeros((G, 1), jnp.float32)

    deg = degt_ref[:, 0:1] + degt_ref[:, 1:2] + 1.0
    dis = lax.rsqrt(deg)
    h2 = (acc_ref[0] + acc_ref[1]) * dis + z2_ref[...] + b2_ref[...]
    pt = (bcol_ref[...] == lax.broadcasted_iota(jnp.int32, (1, G), 1))
    pt = pt.astype(jnp.float32)
    dn = (((0,), (0,)), ((), ()))
    gsum[...] += lax.dot_general(pt, h2, dn,
                                 preferred_element_type=jnp.float32)
    cnt[...] += lax.dot_general(pt, jnp.ones((RB, 1), jnp.float32), dn,
                                preferred_element_type=jnp.float32)

    @pl.when(i == NBLK - 1)
    def _():
        g = gsum[...] / jnp.maximum(cnt[...], 1.0)
        g = jnp.dot(g, wl1_ref[...],
                    preferred_element_type=jnp.float32) + bl1_ref[...]
        g = jnp.dot(g, wl2_ref[...],
                    preferred_element_type=jnp.float32) + bl2_ref[...]
        m = jnp.max(g, axis=0, keepdims=True)
        e = jnp.exp(g - m)
        out_ref[...] = e / jnp.sum(e, axis=0, keepdims=True)


def _t3(acc2, z2, degt, bcol, b2r, Wl1, bl1r, Wl2, bl2r):
    return pl.pallas_call(
        _t3_body,
        grid=(NBLK,),
        in_specs=[
            pl.BlockSpec((NC, RB, H), lambda i: (0, i, 0)),
            pl.BlockSpec((RB, H), lambda i: (i, 0)),
            pl.BlockSpec((RB, 2), lambda i: (i, 0)),
            pl.BlockSpec((RB, 1), lambda i: (i, 0)),
            pl.BlockSpec((1, H), lambda i: (0, 0)),
            pl.BlockSpec((H, 32), lambda i: (0, 0)),
            pl.BlockSpec((1, 32), lambda i: (0, 0)),
            pl.BlockSpec((32, 2), lambda i: (0, 0)),
            pl.BlockSpec((1, 2), lambda i: (0, 0)),
        ],
        out_specs=pl.BlockSpec((G, 2), lambda i: (0, 0)),
        out_shape=jax.ShapeDtypeStruct((G, 2), jnp.float32),
        scratch_shapes=[
            pltpu.VMEM((G, H), jnp.float32),
            pltpu.VMEM((G, 1), jnp.float32),
        ],
    )(acc2, z2, degt, bcol, b2r, Wl1, bl1r, Wl2, bl2r)


def kernel(x, edge_index, batch, W1, b1, W2, b2, Wl1, bl1, Wl2, bl2):
    ei32 = edge_index.astype(jnp.int32)
    src = ei32[0]
    dst = ei32[1]
    pad = EPAD + 8 * CHUNK - E
    srr = jnp.concatenate([src, jnp.zeros((pad,), jnp.int32)])
    srr = srr.reshape(NCH + 8, CHUNK)
    dsr = jnp.concatenate([dst, jnp.full((pad,), N, jnp.int32)])
    dsr = dsr.reshape(NCH + 8, CHUNK)

    degp = _deg_call(ei32)
    degt = degp.T[:N]

    xw1 = _t1a(x, W1)
    y1 = _t1b(xw1, degt)
    acc1 = _scatter_call(y1, srr, dsr)
    y2, z2 = _t2(acc1, xw1, degt, W2, b1.reshape(1, H))
    acc2 = _scatter_call(y2, srr, dsr)
    return _t3(acc2, z2, degt, batch.astype(jnp.int32).reshape(N, 1),
               b2.reshape(1, H), Wl1, bl1.reshape(1, 32),
               Wl2, bl2.reshape(1, 2))


# speedup vs baseline: 41.7945x; 1.0290x over previous
import functools

import jax
import jax.numpy as jnp
from jax import lax
from jax.experimental import pallas as pl
from jax.experimental.pallas import tpu as pltpu
from jax.experimental.pallas import tpu_sc as plsc

N = 10000
E = 320000
DIN = 128
H = 64
G = 128

NC = 2
NS = 16
NW = NC * NS
CHUNK = 128
CPW = 80
NCH = NW * CPW
EPAD = NCH * CHUNK
ROWS = 10240
RPT = ROWS // NS

_MESH = dict(core_axis_name="c", subcore_axis_name="s",
             num_cores=NC, num_subcores=NS)


def _zero_f32_ref(ref, n):
    for k in range(n // 16):
        ref[pl.ds(16 * k, 16)] = jnp.zeros((16,), jnp.float32)


def _deg_body(ei_hbm, out_hbm, srr_out, dsr_out, dbuf, sbuf, lbuf, pbuf,
              dhist, tbuf, tmp, slab, semi, sems):
    c = lax.axis_index("c")
    s = lax.axis_index("s")
    w = c * NS + s
    epw = (E // (NW * CHUNK)) * CHUNK

    pltpu.make_async_copy(ei_hbm.at[1].at[pl.ds(w * epw, epw)], dbuf,
                          semi).start()
    pltpu.make_async_copy(ei_hbm.at[0].at[pl.ds(w * epw, epw)], sbuf,
                          sems).start()

    def zero_hist(j, carry):
        dhist[pl.ds(16 * j, 16)] = jnp.zeros((16,), jnp.float32)
        return carry

    lax.fori_loop(0, ROWS // 16, zero_hist, 0)
    pltpu.make_async_copy(ei_hbm.at[1].at[pl.ds(w * epw, epw)], dbuf,
                          semi).wait()
    pltpu.sync_copy(dbuf, dsr_out.at[pl.ds(w * epw, epw)])
    pltpu.make_async_copy(ei_hbm.at[0].at[pl.ds(w * epw, epw)], sbuf,
                          sems).wait()
    pltpu.sync_copy(sbuf, srr_out.at[pl.ds(w * epw, epw)])

    for k in range(16):
        pbuf[0, pl.ds(16 * k, 16)] = jnp.zeros((16,), jnp.int32)
        pbuf[1, pl.ds(16 * k, 16)] = jnp.full((16,), N, jnp.int32)
    pltpu.sync_copy(pbuf.at[0].at[pl.ds(0, 256)],
                    srr_out.at[pl.ds(E + w * 256, 256)])
    pltpu.sync_copy(pbuf.at[1].at[pl.ds(0, 256)],
                    dsr_out.at[pl.ds(E + w * 256, 256)])

    ones = jnp.ones((16,), jnp.float32)

    def hist(j, carry):
        idx = dbuf[pl.ds(16 * j, 16)]
        plsc.addupdate_scatter(dhist, [idx], ones)
        return carry

    lax.fori_loop(0, epw // 16, hist, 0)

    @pl.when(w < 4)
    def _():
        pltpu.sync_copy(
            ei_hbm.at[1].at[pl.ds(NW * epw + w * CHUNK, CHUNK)], lbuf)
        pltpu.sync_copy(lbuf, dsr_out.at[pl.ds(NW * epw + w * CHUNK, CHUNK)])

        def histl(j, carry):
            idx = lbuf[pl.ds(16 * j, 16)]
            plsc.addupdate_scatter(dhist, [idx], ones)
            return carry

        lax.fori_loop(0, CHUNK // 16, histl, 0)
        pltpu.sync_copy(
            ei_hbm.at[0].at[pl.ds(NW * epw + w * CHUNK, CHUNK)], lbuf)
        pltpu.sync_copy(lbuf, srr_out.at[pl.ds(NW * epw + w * CHUNK, CHUNK)])
        pltpu.sync_copy(
            pbuf.at[0].at[pl.ds(0, CHUNK)],
            srr_out.at[pl.ds(E + NW * 256 + w * CHUNK, CHUNK)])
        pltpu.sync_copy(
            pbuf.at[1].at[pl.ds(0, CHUNK)],
            dsr_out.at[pl.ds(E + NW * 256 + w * CHUNK, CHUNK)])

    pltpu.sync_copy(dhist, slab.at[s])
    plsc.subcore_barrier()
    pltpu.sync_copy(slab.at[:, pl.ds(s * RPT, RPT)], tbuf)

    def red(j, carry):
        o = 16 * j
        v = tbuf[0, pl.ds(o, 16)]
        for k in range(1, NS):
            v = v + tbuf[k, pl.ds(o, 16)]
        tmp[pl.ds(o, 16)] = v
        return carry

    lax.fori_loop(0, RPT // 16, red, 0)
    pltpu.sync_copy(tmp, out_hbm.at[c].at[pl.ds(s * RPT, RPT)])


@functools.partial(jax.jit)
def _deg_call(ei):
    epw = (E // (NW * CHUNK)) * CHUNK
    return pl.kernel(
        _deg_body,
        out_type=[
            jax.ShapeDtypeStruct((NC, ROWS), jnp.float32),
            jax.ShapeDtypeStruct(((NCH + 8) * CHUNK,), jnp.int32),
            jax.ShapeDtypeStruct(((NCH + 8) * CHUNK,), jnp.int32),
        ],
        mesh=plsc.VectorSubcoreMesh(**_MESH),
        scratch_types=[
            pltpu.VMEM((epw,), jnp.int32),
            pltpu.VMEM((epw,), jnp.int32),
            pltpu.VMEM((CHUNK,), jnp.int32),
            pltpu.VMEM((2, 256), jnp.int32),
            pltpu.VMEM((ROWS,), jnp.float32),
            pltpu.VMEM((NS, RPT), jnp.float32),
            pltpu.VMEM((RPT,), jnp.float32),
            pltpu.VMEM_SHARED((NS, ROWS), jnp.float32),
            pltpu.SemaphoreType.DMA,
            pltpu.SemaphoreType.DMA,
        ],
        compiler_params=pltpu.CompilerParams(needs_layout_passes=False),
    )(ei)


def _scatter_body(y_hbm, sr_hbm, dr_hbm, out_hbm, ib0s, ib0d, ib1s, ib1d,
                  rb0, rb1, rb2, rb3, zbuf, acc, ytab,
                  sg0, sg1, sg2, sg3, ss0, ss1, ss2, ss3,
                  si0s, si0d, si1s, si1d, semy):
    c = lax.axis_index("c")
    s = lax.axis_index("s")
    w = c * NS + s

    pltpu.make_async_copy(y_hbm.at[pl.ds(s * (N // NS), N // NS)],
                          ytab.at[pl.ds(s * (N // NS), N // NS)],
                          semy).start()

    for r in range(16):
        for k in range(H // 16):
            zbuf[r, pl.ds(16 * k, 16)] = jnp.zeros((16,), jnp.float32)

    def zloop(j, carry):
        pltpu.sync_copy(zbuf, acc.at[pl.ds(s * RPT + 16 * j, 16)])
        return carry

    lax.fori_loop(0, RPT // 16, zloop, 0)
    pltpu.make_async_copy(y_hbm.at[pl.ds(s * (N // NS), N // NS)],
                          ytab.at[pl.ds(s * (N // NS), N // NS)],
                          semy).wait()
    plsc.subcore_barrier()

    wbase = w * CPW
    rb = (rb0, rb1, rb2, rb3)
    sg = (sg0, sg1, sg2, sg3)

    def gath(ibs, k, r):
        return pltpu.make_async_copy(ytab.at[ibs.at[k]], rb[r], sg[r])

    def lds(ibs, ibd, c, lds_, ldd_):
        return (pltpu.make_async_copy(sr_hbm.at[pl.ds(c, 4)], ibs, lds_),
                pltpu.make_async_copy(dr_hbm.at[pl.ds(c, 4)], ibd, ldd_))

    def quarter(ibXs, ibXd, ibYs, ibYd, c, ldX, ldY):
        gath(ibXs, 0, 0).wait()
        d0 = pltpu.async_copy(rb0, acc.at[ibXd.at[0]], ss0, add=True)
        gath(ibXs, 2, 2).start()
        gath(ibXs, 1, 1).wait()
        d1 = pltpu.async_copy(rb1, acc.at[ibXd.at[1]], ss1, add=True)
        gath(ibXs, 3, 3).start()
        for cp in lds(ibYs, ibYd, c + 4, *ldY):
            cp.wait()
        d0.wait()
        gath(ibYs, 0, 0).start()
        gath(ibXs, 2, 2).wait()
        d2 = pltpu.async_copy(rb2, acc.at[ibXd.at[2]], ss2, add=True)
        d1.wait()
        gath(ibYs, 1, 1).start()
        gath(ibXs, 3, 3).wait()
        d3 = pltpu.async_copy(rb3, acc.at[ibXd.at[3]], ss3, add=True)
        d2.wait()
        d3.wait()
        for cp in lds(ibXs, ibXd, c + 8, *ldX):
            cp.start()

    pltpu.sync_copy(sr_hbm.at[pl.ds(wbase, 4)], ib0s)
    pltpu.sync_copy(dr_hbm.at[pl.ds(wbase, 4)], ib0d)
    for cp in lds(ib1s, ib1d, wbase + 4, si1s, si1d):
        cp.start()
    gath(ib0s, 0, 0).start()
    gath(ib0s, 1, 1).start()

    def body(u, carry):
        c = wbase + 8 * u
        quarter(ib0s, ib0d, ib1s, ib1d, c, (si0s, si0d), (si1s, si1d))
        quarter(ib1s, ib1d, ib0s, ib0d, c + 4, (si1s, si1d), (si0s, si0d))
        return carry

    lax.fori_loop(0, CPW // 8, body, 0)
    gath(ib0s, 0, 0).wait()
    gath(ib0s, 1, 1).wait()
    for cp in lds(ib1s, ib1d, wbase + CPW + 4, si1s, si1d):
        cp.wait()
    plsc.subcore_barrier()
    pltpu.sync_copy(acc.at[pl.ds(s * RPT, RPT)],
                    out_hbm.at[c].at[pl.ds(s * RPT, RPT)])


@functools.partial(jax.jit)
def _scatter_call(y, srr, dsr):
    return pl.kernel(
        _scatter_body,
        out_type=jax.ShapeDtypeStruct((NC, ROWS, H), jnp.float32),
        mesh=plsc.VectorSubcoreMesh(**_MESH),
        scratch_types=(
            [pltpu.VMEM((4, CHUNK), jnp.int32)] * 4
            + [pltpu.VMEM((CHUNK, H), jnp.float32)] * 4
            + [pltpu.VMEM((16, H), jnp.float32),
               pltpu.VMEM_SHARED((ROWS, H), jnp.float32),
               pltpu.VMEM_SHARED((N, H), jnp.float32)]
            + [pltpu.SemaphoreType.DMA] * 13
        ),
        compiler_params=pltpu.CompilerParams(use_tc_tiling_on_sc=False),
    )(y, srr, dsr)


RB = 2000
NBLK = N // RB


def _t1a_body(x_ref, w1_ref, xw_ref):
    xw_ref[...] = jnp.dot(x_ref[...], w1_ref[...],
                          preferred_element_type=jnp.float32)


def _t1a(x, W1):
    return pl.pallas_call(
        _t1a_body,
        grid=(NBLK,),
        in_specs=[
            pl.BlockSpec((RB, DIN), lambda i: (i, 0)),
            pl.BlockSpec((DIN, H), lambda i: (0, 0)),
        ],
        out_specs=pl.BlockSpec((RB, H), lambda i: (i, 0)),
        out_shape=jax.ShapeDtypeStruct((N, H), jnp.float32),
    )(x, W1)


def _t1b_body(xw_ref, degt_ref, y_ref):
    deg = degt_ref[:, 0:1] + degt_ref[:, 1:2] + 1.0
    y_ref[...] = xw_ref[...] * lax.rsqrt(deg)


def _t1b(xw, degt):
    return pl.pallas_call(
        _t1b_body,
        grid=(NBLK,),
        in_specs=[
            pl.BlockSpec((RB, H), lambda i: (i, 0)),
            pl.BlockSpec((RB, 2), lambda i: (i, 0)),
        ],
        out_specs=pl.BlockSpec((RB, H), lambda i: (i, 0)),
        out_shape=jax.ShapeDtypeStruct((N, H), jnp.float32),
    )(xw, degt)


def _t2_body(acc_ref, xw1_ref, degt_ref, w2_ref, b1_ref, y_ref, z_ref):
    deg = degt_ref[:, 0:1] + degt_ref[:, 1:2] + 1.0
    dis = lax.rsqrt(deg)
    a = acc_ref[0] + acc_ref[1]
    h1 = jnp.maximum(a * dis + xw1_ref[...] / deg + b1_ref[...], 0.0)
    xw = jnp.dot(h1, w2_ref[...], preferred_element_type=jnp.float32)
    y_ref[...] = xw * dis
    z_ref[...] = xw / deg


def _t2(acc1, xw1, degt, W2, b1r):
    return pl.pallas_call(
        _t2_body,
        grid=(NBLK,),
        in_specs=[
            pl.BlockSpec((NC, RB, H), lambda i: (0, i, 0)),
            pl.BlockSpec((RB, H), lambda i: (i, 0)),
            pl.BlockSpec((RB, 2), lambda i: (i, 0)),
            pl.BlockSpec((H, H), lambda i: (0, 0)),
            pl.BlockSpec((1, H), lambda i: (0, 0)),
        ],
        out_specs=[
            pl.BlockSpec((RB, H), lambda i: (i, 0)),
            pl.BlockSpec((RB, H), lambda i: (i, 0)),
        ],
        out_shape=[
            jax.ShapeDtypeStruct((N, H), jnp.float32),
            jax.ShapeDtypeStruct((N, H), jnp.float32),
        ],
    )(acc1, xw1, degt, W2, b1r)


def _t3_body(acc_ref, z2_ref, degt_ref, bcol_ref, b2_ref, wl1_ref, bl1_ref,
             wl2_ref, bl2_ref, out_ref, gsum, cnt):
    i = pl.program_id(0)

    @pl.when(i == 0)
    def _():
        gsum[...] = jnp.zeros((G, H), jnp.float32)
        cnt[...] = jnp.zeros((G, 1), jnp.float32)

    deg = degt_ref[:, 0:1] + degt_ref[:, 1:2] + 1.0
    dis = lax.rsqrt(deg)
    h2 = (acc_ref[0] + acc_ref[1]) * dis + z2_ref[...] + b2_ref[...]
    pt = (bcol_ref[...] == lax.broadcasted_iota(jnp.int32, (1, G), 1))
    pt = pt.astype(jnp.float32)
    dn = (((0,), (0,)), ((), ()))
    gsum[...] += lax.dot_general(pt, h2, dn,
                                 preferred_element_type=jnp.float32)
    cnt[...] += lax.dot_general(pt, jnp.ones((RB, 1), jnp.float32), dn,
                                preferred_element_type=jnp.float32)

    @pl.when(i == NBLK - 1)
    def _():
        g = gsum[...] / jnp.maximum(cnt[...], 1.0)
        g = jnp.dot(g, wl1_ref[...],
                    preferred_element_type=jnp.float32) + bl1_ref[...]
        g = jnp.dot(g, wl2_ref[...],
                    preferred_element_type=jnp.float32) + bl2_ref[...]
        m = jnp.max(g, axis=0, keepdims=True)
        e = jnp.exp(g - m)
        out_ref[...] = e / jnp.sum(e, axis=0, keepdims=True)


def _t3(acc2, z2, degt, bcol, b2r, Wl1, bl1r, Wl2, bl2r):
    return pl.pallas_call(
        _t3_body,
        grid=(NBLK,),
        in_specs=[
            pl.BlockSpec((NC, RB, H), lambda i: (0, i, 0)),
            pl.BlockSpec((RB, H), lambda i: (i, 0)),
            pl.BlockSpec((RB, 2), lambda i: (i, 0)),
            pl.BlockSpec((RB, 1), lambda i: (i, 0)),
            pl.BlockSpec((1, H), lambda i: (0, 0)),
            pl.BlockSpec((H, 32), lambda i: (0, 0)),
            pl.BlockSpec((1, 32), lambda i: (0, 0)),
            pl.BlockSpec((32, 2), lambda i: (0, 0)),
            pl.BlockSpec((1, 2), lambda i: (0, 0)),
        ],
        out_specs=pl.BlockSpec((G, 2), lambda i: (0, 0)),
        out_shape=jax.ShapeDtypeStruct((G, 2), jnp.float32),
        scratch_shapes=[
            pltpu.VMEM((G, H), jnp.float32),
            pltpu.VMEM((G, 1), jnp.float32),
        ],
    )(acc2, z2, degt, bcol, b2r, Wl1, bl1r, Wl2, bl2r)


def kernel(x, edge_index, batch, W1, b1, W2, b2, Wl1, bl1, Wl2, bl2):
    ei32 = edge_index.astype(jnp.int32)
    degp, srr, dsr = _deg_call(ei32)
    srr = srr.reshape(NCH + 8, CHUNK)
    dsr = dsr.reshape(NCH + 8, CHUNK)
    degt = degp.T[:N]

    xw1 = _t1a(x, W1)
    y1 = _t1b(xw1, degt)
    acc1 = _scatter_call(y1, srr, dsr)
    y2, z2 = _t2(acc1, xw1, degt, W2, b1.reshape(1, H))
    acc2 = _scatter_call(y2, srr, dsr)
    return _t3(acc2, z2, degt, batch.astype(jnp.int32).reshape(N, 1),
               b2.reshape(1, H), Wl1, bl1.reshape(1, 32),
               Wl2, bl2.reshape(1, 2))
